# Initial kernel scaffold; baseline (speedup 1.0000x reference)
import jax
import jax.numpy as jnp
from jax.experimental import pallas as pl


def kernel(node_features, edge_features, edge_index, shared_node_weights, shared_edge_weights, nW1, nb1, nW2, nb2, nG1, ng1, nG2, ng2, nWw, nbw, eW1, eb1, eW2, eb2, eG1, eg1, eG2, eg2, eWw, ebw):
    raise NotImplementedError("write your pallas kernel here")


# speedup vs baseline: 2.5180x; 2.5180x over previous
import functools

import jax
import jax.numpy as jnp
from jax import lax
from jax.experimental import pallas as pl
from jax.experimental.pallas import tpu as pltpu
from jax.experimental.pallas import tpu_sc as plsc

NN = 10000
NE = 320000
DN = 128
DE = 16
DW = 32

NC = 2
NS = 16
NW = NC * NS
CH = 128
NROW = NE // CH
ROWS_BASE = NROW // NW
ROWS_EXTRA = NROW % NW


def _silu(x):
    return x * jax.nn.sigmoid(x)


def _make_sc_gather(width):
    mesh = plsc.VectorSubcoreMesh(core_axis_name="c", subcore_axis_name="s")

    @functools.partial(
        pl.kernel,
        mesh=mesh,
        out_type=(
            jax.ShapeDtypeStruct((NROW, CH, width), jnp.float32),
            jax.ShapeDtypeStruct((NROW, CH, width), jnp.float32),
        ),
        scratch_types=[
            pltpu.VMEM((2 * CH,), jnp.int32),
            pltpu.VMEM((2 * CH,), jnp.int32),
            pltpu.VMEM((2, CH, width), jnp.float32),
            pltpu.VMEM((2, CH, width), jnp.float32),
            pltpu.SemaphoreType.DMA,
            pltpu.SemaphoreType.DMA,
        ],
    )
    def gather_k(ts_hbm, td_hbm, s_hbm, d_hbm, gs_hbm, gd_hbm,
                 idxs, idxd, rs, rd, gsem, wsem):
        c = lax.axis_index("c")
        s = lax.axis_index("s")
        wid = s * NC + c
        base = ROWS_BASE * wid + jnp.minimum(wid, ROWS_EXTRA)

        def do_pair(row, first):
            pltpu.sync_copy(s_hbm.at[pl.ds(row * CH, 2 * CH)], idxs)
            pltpu.sync_copy(d_hbm.at[pl.ds(row * CH, 2 * CH)], idxd)
            @pl.when(jnp.logical_not(first))
            def _():
                pltpu.make_async_copy(rs, gs_hbm.at[pl.ds(row, 2)], wsem).wait()
                pltpu.make_async_copy(rd, gd_hbm.at[pl.ds(row, 2)], wsem).wait()
            d0 = pltpu.async_copy(ts_hbm.at[idxs.at[pl.ds(0, CH)]], rs.at[0], gsem)
            d1 = pltpu.async_copy(ts_hbm.at[idxs.at[pl.ds(CH, CH)]], rs.at[1], gsem)
            d2 = pltpu.async_copy(td_hbm.at[idxd.at[pl.ds(0, CH)]], rd.at[0], gsem)
            d3 = pltpu.async_copy(td_hbm.at[idxd.at[pl.ds(CH, CH)]], rd.at[1], gsem)
            d0.wait(); d1.wait(); d2.wait(); d3.wait()
            pltpu.async_copy(rs, gs_hbm.at[pl.ds(row, 2)], wsem)
            pltpu.async_copy(rd, gd_hbm.at[pl.ds(row, 2)], wsem)

        def body(i, _):
            do_pair(base + 2 * i, i == 0)
            return 0

        lax.fori_loop(0, ROWS_BASE // 2, body, 0)
        pltpu.make_async_copy(rs, gs_hbm.at[pl.ds(base, 2)], wsem).wait()
        pltpu.make_async_copy(rd, gd_hbm.at[pl.ds(base, 2)], wsem).wait()

        @pl.when(wid < ROWS_EXTRA)
        def _():
            row = base + ROWS_BASE
            pltpu.sync_copy(s_hbm.at[pl.ds(row * CH, CH)], idxs.at[pl.ds(0, CH)])
            pltpu.sync_copy(d_hbm.at[pl.ds(row * CH, CH)], idxd.at[pl.ds(0, CH)])
            d0 = pltpu.async_copy(ts_hbm.at[idxs.at[pl.ds(0, CH)]], rs.at[0], gsem)
            d1 = pltpu.async_copy(td_hbm.at[idxd.at[pl.ds(0, CH)]], rd.at[0], gsem)
            d0.wait(); d1.wait()
            pltpu.sync_copy(rs.at[pl.ds(0, 1)], gs_hbm.at[pl.ds(row, 1)])
            pltpu.sync_copy(rd.at[pl.ds(0, 1)], gd_hbm.at[pl.ds(row, 1)])

    return gather_k


def _make_sc_scatter():
    mesh = plsc.VectorSubcoreMesh(core_axis_name="c", subcore_axis_name="s")
    ZR = 624

    @functools.partial(
        pl.kernel,
        mesh=mesh,
        out_type=jax.ShapeDtypeStruct((NC, NN, DN), jnp.float32),
        scratch_types=[
            pltpu.VMEM((CH,), jnp.int32),
            pltpu.VMEM((CH,), jnp.int32),
            pltpu.VMEM((2, CH, DN), jnp.float32),
            pltpu.VMEM_SHARED((NN, DN), jnp.float32),
            pltpu.SemaphoreType.DMA,
        ],
    )
    def scatter_k(u_hbm, d_hbm, z_hbm, out_hbm, idx0, idx1, upd, acc, sem):
        c = lax.axis_index("c")
        s = lax.axis_index("s")
        wid = s * NC + c
        base = ROWS_BASE * wid + jnp.minimum(wid, ROWS_EXTRA)

        pltpu.sync_copy(z_hbm.at[pl.ds(s * ZR, ZR)], acc.at[pl.ds(s * ZR, ZR)])
        @pl.when(s == NS - 1)
        def _():
            tail = NN - NS * ZR
            pltpu.sync_copy(z_hbm.at[pl.ds(NS * ZR, tail)],
                            acc.at[pl.ds(NS * ZR, tail)])
        plsc.subcore_barrier()

        def body(i, _):
            row = base + 2 * i
            pltpu.sync_copy(d_hbm.at[pl.ds(row * CH, CH)], idx0)
            pltpu.sync_copy(d_hbm.at[pl.ds((row + 1) * CH, CH)], idx1)
            pltpu.sync_copy(u_hbm.at[pl.ds(row, 2)], upd)
            a0 = pltpu.async_copy(upd.at[0], acc.at[idx0], sem, add=True)
            a1 = pltpu.async_copy(upd.at[1], acc.at[idx1], sem, add=True)
            a0.wait(); a1.wait()
            return 0

        lax.fori_loop(0, ROWS_BASE // 2, body, 0)

        @pl.when(wid < ROWS_EXTRA)
        def _():
            row = base + ROWS_BASE
            pltpu.sync_copy(d_hbm.at[pl.ds(row * CH, CH)], idx0)
            pltpu.sync_copy(u_hbm.at[pl.ds(row, 1)], upd.at[pl.ds(0, 1)])
            pltpu.sync_copy(upd.at[0], acc.at[idx0], add=True)

        plsc.subcore_barrier()
        pltpu.sync_copy(acc.at[pl.ds(s * ZR, ZR)], out_hbm.at[c, pl.ds(s * ZR, ZR)])
        @pl.when(s == NS - 1)
        def _():
            tail = NN - NS * ZR
            pltpu.sync_copy(acc.at[pl.ds(NS * ZR, tail)],
                            out_hbm.at[c, pl.ds(NS * ZR, tail)])

    return scatter_k


_sc_gather_dn = _make_sc_gather(DN)
_sc_scatter = _make_sc_scatter()


def _node_msg_body(gs, gd, ef, snw, w1s, w1e, w1d, b1, w2, b2,
                   g1s, g1e, g1d, c1, g2, c2, ww, bw, out):
    f32 = jnp.float32
    xs = gs[...]
    xd = gd[...]
    e = ef[...]
    pre_h = (jnp.dot(xs, w1s[...], preferred_element_type=f32)
             + jnp.dot(e, w1e[...], preferred_element_type=f32)
             + jnp.dot(xd, w1d[...], preferred_element_type=f32) + b1[...])
    pre_g = (jnp.dot(xs, g1s[...], preferred_element_type=f32)
             + jnp.dot(e, g1e[...], preferred_element_type=f32)
             + jnp.dot(xd, g1d[...], preferred_element_type=f32) + c1[...])
    h2 = _silu(jnp.dot(_silu(pre_h), w2[...], preferred_element_type=f32) + b2[...])
    gg = jax.nn.sigmoid(jnp.dot(_silu(pre_g), g2[...], preferred_element_type=f32) + c2[...])
    w = jax.nn.sigmoid(jnp.dot(snw[...], ww[...], preferred_element_type=f32) + bw[...])
    out[...] = h2 * gg * w


def _tc_node_msg(gs, gd, ef, snw, w1s, w1e, w1d, b1, w2, b2,
                 g1s, g1e, g1d, c1, g2, c2, ww, bw):
    BN = 1280
    grid = NE // BN
    row = lambda i: (i, 0)
    full = lambda i: (0, 0)

    def wspec(a):
        return pl.BlockSpec(a.shape, full)

    return pl.pallas_call(
        _node_msg_body,
        grid=(grid,),
        in_specs=[
            pl.BlockSpec((BN, DN), row),
            pl.BlockSpec((BN, DN), row),
            pl.BlockSpec((BN, DE), row),
            pl.BlockSpec((BN, DW), row),
            wspec(w1s), wspec(w1e), wspec(w1d), wspec(b1),
            wspec(w2), wspec(b2),
            wspec(g1s), wspec(g1e), wspec(g1d), wspec(c1),
            wspec(g2), wspec(c2), wspec(ww), wspec(bw),
        ],
        out_specs=pl.BlockSpec((BN, DN), row),
        out_shape=jax.ShapeDtypeStruct((NE, DN), jnp.float32),
        compiler_params=pltpu.CompilerParams(
            dimension_semantics=("arbitrary",)),
    )(gs, gd, ef, snw, w1s, w1e, w1d, b1, w2, b2,
      g1s, g1e, g1d, c1, g2, c2, ww, bw)


def _node_out_body(nf, a0, a1, ew1s, eg1s, ew1d, eg1d, nn_out, te_out):
    f32 = jnp.float32
    nn = nf[...] + a0[...] + a1[...]
    nn_out[...] = nn
    te_out[...] = jnp.concatenate(
        [jnp.dot(nn, ew1s[...], preferred_element_type=f32),
         jnp.dot(nn, eg1s[...], preferred_element_type=f32),
         jnp.dot(nn, ew1d[...], preferred_element_type=f32),
         jnp.dot(nn, eg1d[...], preferred_element_type=f32)], axis=1)


def _tc_node_out(nf, a0, a1, ew1s, eg1s, ew1d, eg1d):
    BR = 1000
    grid = NN // BR
    row = lambda i: (i, 0)
    full = lambda i: (0, 0)

    def wspec(a):
        return pl.BlockSpec(a.shape, full)

    return pl.pallas_call(
        _node_out_body,
        grid=(grid,),
        in_specs=[
            pl.BlockSpec((BR, DN), row),
            pl.BlockSpec((BR, DN), row),
            pl.BlockSpec((BR, DN), row),
            wspec(ew1s), wspec(eg1s), wspec(ew1d), wspec(eg1d),
        ],
        out_specs=[
            pl.BlockSpec((BR, DN), row),
            pl.BlockSpec((BR, DN), row),
        ],
        out_shape=[
            jax.ShapeDtypeStruct((NN, DN), jnp.float32),
            jax.ShapeDtypeStruct((NN, DN), jnp.float32),
        ],
        compiler_params=pltpu.CompilerParams(
            dimension_semantics=("arbitrary",)),
    )(nf, a0, a1, ew1s, eg1s, ew1d, eg1d)


def _edge_out_body(ges, ged, ef, sew, ew1e, eg1e, eb1, eg1, ew2, eb2,
                   eg2, eg2b, eww, ebw, out):
    f32 = jnp.float32
    e = ef[...]
    gs = ges[...]
    gd = ged[...]
    pre_h = (gs[:, :32] + gd[:, 64:96]
             + jnp.dot(e, ew1e[...], preferred_element_type=f32) + eb1[...])
    pre_g = (gs[:, 32:64] + gd[:, 96:128]
             + jnp.dot(e, eg1e[...], preferred_element_type=f32) + eg1[...])
    h2 = _silu(jnp.dot(_silu(pre_h), ew2[...], preferred_element_type=f32) + eb2[...])
    g2 = jax.nn.sigmoid(jnp.dot(_silu(pre_g), eg2[...], preferred_element_type=f32) + eg2b[...])
    ew = jax.nn.sigmoid(jnp.dot(sew[...], eww[...], preferred_element_type=f32) + ebw[...])
    out[...] = e + h2 * g2 * ew


def _tc_edge_out(ges, ged, ef, sew, ew1e, eg1e, eb1, eg1, ew2, eb2,
                 eg2, eg2b, eww, ebw):
    BN = 2000
    grid = NE // BN
    row = lambda i: (i, 0)
    full = lambda i: (0, 0)

    def wspec(a):
        return pl.BlockSpec(a.shape, full)

    return pl.pallas_call(
        _edge_out_body,
        grid=(grid,),
        in_specs=[
            pl.BlockSpec((BN, DN), row),
            pl.BlockSpec((BN, DN), row),
            pl.BlockSpec((BN, DE), row),
            pl.BlockSpec((BN, DW), row),
            wspec(ew1e), wspec(eg1e), wspec(eb1), wspec(eg1),
            wspec(ew2), wspec(eb2), wspec(eg2), wspec(eg2b),
            wspec(eww), wspec(ebw),
        ],
        out_specs=pl.BlockSpec((BN, DE), row),
        out_shape=jax.ShapeDtypeStruct((NE, DE), jnp.float32),
        compiler_params=pltpu.CompilerParams(
            dimension_semantics=("arbitrary",)),
    )(ges, ged, ef, sew, ew1e, eg1e, eb1, eg1, ew2, eb2, eg2, eg2b, eww, ebw)


def kernel(node_features, edge_features, edge_index, shared_node_weights,
           shared_edge_weights, nW1, nb1, nW2, nb2, nG1, ng1, nG2, ng2,
           nWw, nbw, eW1, eb1, eW2, eb2, eG1, eg1, eG2, eg2, eWw, ebw):
    f32 = jnp.float32
    src = edge_index[0]
    dst = edge_index[1]

    gs3, gd3 = _sc_gather_dn(node_features, node_features, src, dst)
    gs = jnp.reshape(gs3, (NE, DN))
    gd = jnp.reshape(gd3, (NE, DN))

    contrib = _tc_node_msg(
        gs, gd, edge_features, shared_node_weights,
        nW1[:DN], nW1[DN:DN + DE], nW1[DN + DE:], nb1.reshape(1, DN),
        nW2, nb2.reshape(1, DN),
        nG1[:DN], nG1[DN:DN + DE], nG1[DN + DE:], ng1.reshape(1, DN),
        nG2, ng2.reshape(1, DN), nWw, nbw.reshape(1, DN))

    zeros = jnp.zeros((NN, DN), f32)
    aggp = _sc_scatter(jnp.reshape(contrib, (NROW, CH, DN)), dst, zeros)

    nn, te = _tc_node_out(node_features, aggp[0], aggp[1],
                          eW1[:DN], eG1[:DN], eW1[DN + DE:], eG1[DN + DE:])

    ges3, ged3 = _sc_gather_dn(te, te, src, dst)
    ges = jnp.reshape(ges3, (NE, DN))
    ged = jnp.reshape(ged3, (NE, DN))

    new_edges = _tc_edge_out(
        ges, ged, edge_features, shared_edge_weights,
        eW1[DN:DN + DE], eG1[DN:DN + DE], eb1.reshape(1, 32), eg1.reshape(1, 32),
        eW2, eb2.reshape(1, DE), eG2, eg2.reshape(1, DE),
        eWw, ebw.reshape(1, DE))

    return nn, new_edges


# speedup vs baseline: 3.0454x; 1.2094x over previous
import functools

import jax
import jax.numpy as jnp
from jax import lax
from jax.experimental import pallas as pl
from jax.experimental.pallas import tpu as pltpu
from jax.experimental.pallas import tpu_sc as plsc

NN = 10000
NE = 320000
DN = 128
DE = 16
DW = 32

NC = 2
NS = 16
NW = NC * NS
CH = 128
NROW = NE // CH
ROWS_BASE = NROW // NW
ROWS_EXTRA = NROW % NW


def _silu(x):
    return x * jax.nn.sigmoid(x)


def _make_sc_gather(width):
    mesh = plsc.VectorSubcoreMesh(core_axis_name="c", subcore_axis_name="s")

    @functools.partial(
        pl.kernel,
        mesh=mesh,
        out_type=(
            jax.ShapeDtypeStruct((NROW, CH, width), jnp.float32),
            jax.ShapeDtypeStruct((NROW, CH, width), jnp.float32),
        ),
        scratch_types=[
            pltpu.VMEM((2 * CH,), jnp.int32),
            pltpu.VMEM((2 * CH,), jnp.int32),
            pltpu.VMEM((2, CH, width), jnp.float32),
            pltpu.VMEM((2, CH, width), jnp.float32),
            pltpu.SemaphoreType.DMA,
            pltpu.SemaphoreType.DMA,
        ],
    )
    def gather_k(ts_hbm, td_hbm, s_hbm, d_hbm, gs_hbm, gd_hbm,
                 idxs, idxd, rs, rd, gsem, wsem):
        c = lax.axis_index("c")
        s = lax.axis_index("s")
        wid = s * NC + c
        base = ROWS_BASE * wid + jnp.minimum(wid, ROWS_EXTRA)

        def do_pair(row, first):
            pltpu.sync_copy(s_hbm.at[pl.ds(row * CH, 2 * CH)], idxs)
            pltpu.sync_copy(d_hbm.at[pl.ds(row * CH, 2 * CH)], idxd)
            @pl.when(jnp.logical_not(first))
            def _():
                pltpu.make_async_copy(rs, gs_hbm.at[pl.ds(row, 2)], wsem).wait()
                pltpu.make_async_copy(rd, gd_hbm.at[pl.ds(row, 2)], wsem).wait()
            d0 = pltpu.async_copy(ts_hbm.at[idxs.at[pl.ds(0, CH)]], rs.at[0], gsem)
            d1 = pltpu.async_copy(ts_hbm.at[idxs.at[pl.ds(CH, CH)]], rs.at[1], gsem)
            d2 = pltpu.async_copy(td_hbm.at[idxd.at[pl.ds(0, CH)]], rd.at[0], gsem)
            d3 = pltpu.async_copy(td_hbm.at[idxd.at[pl.ds(CH, CH)]], rd.at[1], gsem)
            d0.wait(); d1.wait(); d2.wait(); d3.wait()
            pltpu.async_copy(rs, gs_hbm.at[pl.ds(row, 2)], wsem)
            pltpu.async_copy(rd, gd_hbm.at[pl.ds(row, 2)], wsem)

        def body(i, _):
            do_pair(base + 2 * i, i == 0)
            return 0

        lax.fori_loop(0, ROWS_BASE // 2, body, 0)
        pltpu.make_async_copy(rs, gs_hbm.at[pl.ds(base, 2)], wsem).wait()
        pltpu.make_async_copy(rd, gd_hbm.at[pl.ds(base, 2)], wsem).wait()

        @pl.when(wid < ROWS_EXTRA)
        def _():
            row = base + ROWS_BASE
            pltpu.sync_copy(s_hbm.at[pl.ds(row * CH, CH)], idxs.at[pl.ds(0, CH)])
            pltpu.sync_copy(d_hbm.at[pl.ds(row * CH, CH)], idxd.at[pl.ds(0, CH)])
            d0 = pltpu.async_copy(ts_hbm.at[idxs.at[pl.ds(0, CH)]], rs.at[0], gsem)
            d1 = pltpu.async_copy(td_hbm.at[idxd.at[pl.ds(0, CH)]], rd.at[0], gsem)
            d0.wait(); d1.wait()
            pltpu.sync_copy(rs.at[pl.ds(0, 1)], gs_hbm.at[pl.ds(row, 1)])
            pltpu.sync_copy(rd.at[pl.ds(0, 1)], gd_hbm.at[pl.ds(row, 1)])

    return gather_k


def _make_sc_scatter():
    mesh = plsc.VectorSubcoreMesh(core_axis_name="c", subcore_axis_name="s")
    ZR = 624

    @functools.partial(
        pl.kernel,
        mesh=mesh,
        out_type=jax.ShapeDtypeStruct((NC, NN, DN), jnp.float32),
        scratch_types=[
            pltpu.VMEM((CH,), jnp.int32),
            pltpu.VMEM((CH,), jnp.int32),
            pltpu.VMEM((2, CH, DN), jnp.float32),
            pltpu.VMEM_SHARED((NN, DN), jnp.float32),
            pltpu.SemaphoreType.DMA,
        ],
    )
    def scatter_k(u_hbm, d_hbm, z_hbm, out_hbm, idx0, idx1, upd, acc, sem):
        c = lax.axis_index("c")
        s = lax.axis_index("s")
        wid = s * NC + c
        base = ROWS_BASE * wid + jnp.minimum(wid, ROWS_EXTRA)

        pltpu.sync_copy(z_hbm.at[pl.ds(s * ZR, ZR)], acc.at[pl.ds(s * ZR, ZR)])
        @pl.when(s == NS - 1)
        def _():
            tail = NN - NS * ZR
            pltpu.sync_copy(z_hbm.at[pl.ds(NS * ZR, tail)],
                            acc.at[pl.ds(NS * ZR, tail)])
        plsc.subcore_barrier()

        def body(i, _):
            row = base + 2 * i
            pltpu.sync_copy(d_hbm.at[pl.ds(row * CH, CH)], idx0)
            pltpu.sync_copy(d_hbm.at[pl.ds((row + 1) * CH, CH)], idx1)
            pltpu.sync_copy(u_hbm.at[pl.ds(row, 2)], upd)
            a0 = pltpu.async_copy(upd.at[0], acc.at[idx0], sem, add=True)
            a1 = pltpu.async_copy(upd.at[1], acc.at[idx1], sem, add=True)
            a0.wait(); a1.wait()
            return 0

        lax.fori_loop(0, ROWS_BASE // 2, body, 0)

        @pl.when(wid < ROWS_EXTRA)
        def _():
            row = base + ROWS_BASE
            pltpu.sync_copy(d_hbm.at[pl.ds(row * CH, CH)], idx0)
            pltpu.sync_copy(u_hbm.at[pl.ds(row, 1)], upd.at[pl.ds(0, 1)])
            pltpu.sync_copy(upd.at[0], acc.at[idx0], add=True)

        plsc.subcore_barrier()
        pltpu.sync_copy(acc.at[pl.ds(s * ZR, ZR)], out_hbm.at[c, pl.ds(s * ZR, ZR)])
        @pl.when(s == NS - 1)
        def _():
            tail = NN - NS * ZR
            pltpu.sync_copy(acc.at[pl.ds(NS * ZR, tail)],
                            out_hbm.at[c, pl.ds(NS * ZR, tail)])

    return scatter_k


_sc_gather_dn = _make_sc_gather(DN)
_sc_scatter = _make_sc_scatter()


def _dot0(a, b):
    return lax.dot_general(a, b, (((0,), (0,)), ((), ())),
                           preferred_element_type=jnp.float32)


def _node_msg_body(gs, gd, eft, snwt, w1s, w1e, w1d, b1, w2, b2,
                   g1s, g1e, g1d, c1, g2, c2, ww, bw, out):
    f32 = jnp.float32
    xs = gs[...]
    xd = gd[...]
    et = eft[...]
    pre_h = (jnp.dot(xs, w1s[...], preferred_element_type=f32)
             + _dot0(et, w1e[...])
             + jnp.dot(xd, w1d[...], preferred_element_type=f32) + b1[...])
    pre_g = (jnp.dot(xs, g1s[...], preferred_element_type=f32)
             + _dot0(et, g1e[...])
             + jnp.dot(xd, g1d[...], preferred_element_type=f32) + c1[...])
    h2 = _silu(jnp.dot(_silu(pre_h), w2[...], preferred_element_type=f32) + b2[...])
    gg = jax.nn.sigmoid(jnp.dot(_silu(pre_g), g2[...], preferred_element_type=f32) + c2[...])
    w = jax.nn.sigmoid(_dot0(snwt[...], ww[...]) + bw[...])
    out[...] = h2 * gg * w


def _tc_node_msg(gs, gd, eft, snwt, w1s, w1e, w1d, b1, w2, b2,
                 g1s, g1e, g1d, c1, g2, c2, ww, bw):
    BN = 1280
    grid = NE // BN
    row = lambda i: (i, 0)
    col = lambda i: (0, i)
    full = lambda i: (0, 0)

    def wspec(a):
        return pl.BlockSpec(a.shape, full)

    return pl.pallas_call(
        _node_msg_body,
        grid=(grid,),
        in_specs=[
            pl.BlockSpec((BN, DN), row),
            pl.BlockSpec((BN, DN), row),
            pl.BlockSpec((DE, BN), col),
            pl.BlockSpec((DW, BN), col),
            wspec(w1s), wspec(w1e), wspec(w1d), wspec(b1),
            wspec(w2), wspec(b2),
            wspec(g1s), wspec(g1e), wspec(g1d), wspec(c1),
            wspec(g2), wspec(c2), wspec(ww), wspec(bw),
        ],
        out_specs=pl.BlockSpec((BN, DN), row),
        out_shape=jax.ShapeDtypeStruct((NE, DN), jnp.float32),
        compiler_params=pltpu.CompilerParams(
            dimension_semantics=("arbitrary",)),
    )(gs, gd, eft, snwt, w1s, w1e, w1d, b1, w2, b2,
      g1s, g1e, g1d, c1, g2, c2, ww, bw)


def _node_out_body(nf, a0, a1, ew1s, eg1s, ew1d, eg1d, nn_out, te_out):
    f32 = jnp.float32
    nn = nf[...] + a0[...] + a1[...]
    nn_out[...] = nn
    te_out[...] = jnp.concatenate(
        [jnp.dot(nn, ew1s[...], preferred_element_type=f32),
         jnp.dot(nn, eg1s[...], preferred_element_type=f32),
         jnp.dot(nn, ew1d[...], preferred_element_type=f32),
         jnp.dot(nn, eg1d[...], preferred_element_type=f32)], axis=1)


def _tc_node_out(nf, a0, a1, ew1s, eg1s, ew1d, eg1d):
    BR = 1000
    grid = NN // BR
    row = lambda i: (i, 0)
    full = lambda i: (0, 0)

    def wspec(a):
        return pl.BlockSpec(a.shape, full)

    return pl.pallas_call(
        _node_out_body,
        grid=(grid,),
        in_specs=[
            pl.BlockSpec((BR, DN), row),
            pl.BlockSpec((BR, DN), row),
            pl.BlockSpec((BR, DN), row),
            wspec(ew1s), wspec(eg1s), wspec(ew1d), wspec(eg1d),
        ],
        out_specs=[
            pl.BlockSpec((BR, DN), row),
            pl.BlockSpec((BR, DN), row),
        ],
        out_shape=[
            jax.ShapeDtypeStruct((NN, DN), jnp.float32),
            jax.ShapeDtypeStruct((NN, DN), jnp.float32),
        ],
        compiler_params=pltpu.CompilerParams(
            dimension_semantics=("arbitrary",)),
    )(nf, a0, a1, ew1s, eg1s, ew1d, eg1d)


def _edge_out_body(ges, ged, eft, sewt, ew1e, eg1e, eb1, eg1, ew2, eb2,
                   eg2, eg2b, eww, ebw, out):
    f32 = jnp.float32
    et = eft[...]
    gs = ges[...]
    gd = ged[...]
    pre_h = (gs[:, :32] + gd[:, 64:96] + _dot0(et, ew1e[...]) + eb1[...])
    pre_g = (gs[:, 32:64] + gd[:, 96:128] + _dot0(et, eg1e[...]) + eg1[...])
    h2 = _silu(jnp.dot(_silu(pre_h), ew2[...], preferred_element_type=f32) + eb2[...])
    g2 = jax.nn.sigmoid(jnp.dot(_silu(pre_g), eg2[...], preferred_element_type=f32) + eg2b[...])
    ew = jax.nn.sigmoid(_dot0(sewt[...], eww[...]) + ebw[...])
    msg = h2 * g2 * ew
    out[...] = et + jnp.swapaxes(msg, 0, 1)


def _tc_edge_out(ges, ged, eft, sewt, ew1e, eg1e, eb1, eg1, ew2, eb2,
                 eg2, eg2b, eww, ebw):
    BN = 2560
    grid = NE // BN
    row = lambda i: (i, 0)
    col = lambda i: (0, i)
    full = lambda i: (0, 0)

    def wspec(a):
        return pl.BlockSpec(a.shape, full)

    return pl.pallas_call(
        _edge_out_body,
        grid=(grid,),
        in_specs=[
            pl.BlockSpec((BN, DN), row),
            pl.BlockSpec((BN, DN), row),
            pl.BlockSpec((DE, BN), col),
            pl.BlockSpec((DW, BN), col),
            wspec(ew1e), wspec(eg1e), wspec(eb1), wspec(eg1),
            wspec(ew2), wspec(eb2), wspec(eg2), wspec(eg2b),
            wspec(eww), wspec(ebw),
        ],
        out_specs=pl.BlockSpec((DE, BN), col),
        out_shape=jax.ShapeDtypeStruct((DE, NE), jnp.float32),
        compiler_params=pltpu.CompilerParams(
            dimension_semantics=("arbitrary",)),
    )(ges, ged, eft, sewt, ew1e, eg1e, eb1, eg1, ew2, eb2, eg2, eg2b, eww, ebw)


def kernel(node_features, edge_features, edge_index, shared_node_weights,
           shared_edge_weights, nW1, nb1, nW2, nb2, nG1, ng1, nG2, ng2,
           nWw, nbw, eW1, eb1, eW2, eb2, eG1, eg1, eG2, eg2, eWw, ebw):
    f32 = jnp.float32
    src = edge_index[0]
    dst = edge_index[1]
    eft = jnp.swapaxes(edge_features, 0, 1)
    snwt = jnp.swapaxes(shared_node_weights, 0, 1)
    sewt = jnp.swapaxes(shared_edge_weights, 0, 1)

    gs3, gd3 = _sc_gather_dn(node_features, node_features, src, dst)
    gs = jnp.reshape(gs3, (NE, DN))
    gd = jnp.reshape(gd3, (NE, DN))

    contrib = _tc_node_msg(
        gs, gd, eft, snwt,
        nW1[:DN], nW1[DN:DN + DE], nW1[DN + DE:], nb1.reshape(1, DN),
        nW2, nb2.reshape(1, DN),
        nG1[:DN], nG1[DN:DN + DE], nG1[DN + DE:], ng1.reshape(1, DN),
        nG2, ng2.reshape(1, DN), nWw, nbw.reshape(1, DN))

    zeros = jnp.zeros((NN, DN), f32)
    aggp = _sc_scatter(jnp.reshape(contrib, (NROW, CH, DN)), dst, zeros)

    nn, te = _tc_node_out(node_features, aggp[0], aggp[1],
                          eW1[:DN], eG1[:DN], eW1[DN + DE:], eG1[DN + DE:])

    ges3, ged3 = _sc_gather_dn(te, te, src, dst)
    ges = jnp.reshape(ges3, (NE, DN))
    ged = jnp.reshape(ged3, (NE, DN))

    new_edges_t = _tc_edge_out(
        ges, ged, eft, sewt,
        eW1[DN:DN + DE], eG1[DN:DN + DE], eb1.reshape(1, 32), eg1.reshape(1, 32),
        eW2, eb2.reshape(1, DE), eG2, eg2.reshape(1, DE),
        eWw, ebw.reshape(1, DE))

    return nn, jnp.swapaxes(new_edges_t, 0, 1)


# speedup vs baseline: 3.4867x; 1.1449x over previous
import functools

import jax
import jax.numpy as jnp
from jax import lax
from jax.experimental import pallas as pl
from jax.experimental.pallas import tpu as pltpu
from jax.experimental.pallas import tpu_sc as plsc

NN = 10000
NE = 320000
DN = 128
DE = 16
DW = 32

NC = 2
NS = 16
NW = NC * NS
CH = 128
NROW = NE // CH
ROWS_BASE = NROW // NW
ROWS_EXTRA = NROW % NW


def _silu(x):
    return x * jax.nn.sigmoid(x)


SLICES = ((0, 640), (640, 640), (1280, 640), (1920, 580))


def _make_sc_gather(width, row0, nrows):
    mesh = plsc.VectorSubcoreMesh(core_axis_name="c", subcore_axis_name="s")
    rows_base = nrows // NW
    rows_extra = nrows % NW
    assert rows_base % 2 == 0

    @functools.partial(
        pl.kernel,
        mesh=mesh,
        out_type=(
            jax.ShapeDtypeStruct((nrows, CH, width), jnp.float32),
            jax.ShapeDtypeStruct((nrows, CH, width), jnp.float32),
        ),
        scratch_types=[
            pltpu.VMEM((2 * CH,), jnp.int32),
            pltpu.VMEM((2 * CH,), jnp.int32),
            pltpu.VMEM((2, CH, width), jnp.float32),
            pltpu.VMEM((2, CH, width), jnp.float32),
            pltpu.SemaphoreType.DMA,
            pltpu.SemaphoreType.DMA,
        ],
    )
    def gather_k(tab_hbm, s_hbm, d_hbm, gs_hbm, gd_hbm,
                 idxs, idxd, rs, rd, gsem, wsem):
        c = lax.axis_index("c")
        s = lax.axis_index("s")
        wid = s * NC + c
        base = rows_base * wid + jnp.minimum(wid, rows_extra)

        def do_pair(row, first):
            grow = row0 + row
            pltpu.sync_copy(s_hbm.at[pl.ds(grow * CH, 2 * CH)], idxs)
            pltpu.sync_copy(d_hbm.at[pl.ds(grow * CH, 2 * CH)], idxd)
            @pl.when(jnp.logical_not(first))
            def _():
                pltpu.make_async_copy(rs, gs_hbm.at[pl.ds(row, 2)], wsem).wait()
                pltpu.make_async_copy(rd, gd_hbm.at[pl.ds(row, 2)], wsem).wait()
            d0 = pltpu.async_copy(tab_hbm.at[idxs.at[pl.ds(0, CH)]], rs.at[0], gsem)
            d1 = pltpu.async_copy(tab_hbm.at[idxs.at[pl.ds(CH, CH)]], rs.at[1], gsem)
            d2 = pltpu.async_copy(tab_hbm.at[idxd.at[pl.ds(0, CH)]], rd.at[0], gsem)
            d3 = pltpu.async_copy(tab_hbm.at[idxd.at[pl.ds(CH, CH)]], rd.at[1], gsem)
            d0.wait(); d1.wait(); d2.wait(); d3.wait()
            pltpu.async_copy(rs, gs_hbm.at[pl.ds(row, 2)], wsem)
            pltpu.async_copy(rd, gd_hbm.at[pl.ds(row, 2)], wsem)

        def body(i, _):
            do_pair(base + 2 * i, i == 0)
            return 0

        lax.fori_loop(0, rows_base // 2, body, 0)
        pltpu.make_async_copy(rs, gs_hbm.at[pl.ds(base, 2)], wsem).wait()
        pltpu.make_async_copy(rd, gd_hbm.at[pl.ds(base, 2)], wsem).wait()

        if rows_extra:
            @pl.when(wid < rows_extra)
            def _():
                row = base + rows_base
                grow = row0 + row
                pltpu.sync_copy(s_hbm.at[pl.ds(grow * CH, CH)],
                                idxs.at[pl.ds(0, CH)])
                pltpu.sync_copy(d_hbm.at[pl.ds(grow * CH, CH)],
                                idxd.at[pl.ds(0, CH)])
                d0 = pltpu.async_copy(tab_hbm.at[idxs.at[pl.ds(0, CH)]],
                                      rs.at[0], gsem)
                d1 = pltpu.async_copy(tab_hbm.at[idxd.at[pl.ds(0, CH)]],
                                      rd.at[0], gsem)
                d0.wait(); d1.wait()
                pltpu.sync_copy(rs.at[pl.ds(0, 1)], gs_hbm.at[pl.ds(row, 1)])
                pltpu.sync_copy(rd.at[pl.ds(0, 1)], gd_hbm.at[pl.ds(row, 1)])

    return gather_k


def _make_sc_scatter():
    mesh = plsc.VectorSubcoreMesh(core_axis_name="c", subcore_axis_name="s")
    ZR = 624

    @functools.partial(
        pl.kernel,
        mesh=mesh,
        out_type=jax.ShapeDtypeStruct((NC, NN, DN), jnp.float32),
        scratch_types=[
            pltpu.VMEM((CH,), jnp.int32),
            pltpu.VMEM((CH,), jnp.int32),
            pltpu.VMEM((2, CH, DN), jnp.float32),
            pltpu.VMEM_SHARED((NN, DN), jnp.float32),
            pltpu.SemaphoreType.DMA,
        ],
    )
    def scatter_k(u0_hbm, u1_hbm, u2_hbm, u3_hbm, d_hbm, z_hbm, out_hbm,
                  idx0, idx1, upd, acc, sem):
        c = lax.axis_index("c")
        s = lax.axis_index("s")
        wid = s * NC + c

        pltpu.sync_copy(z_hbm.at[pl.ds(s * ZR, ZR)], acc.at[pl.ds(s * ZR, ZR)])
        @pl.when(s == NS - 1)
        def _():
            tail = NN - NS * ZR
            pltpu.sync_copy(z_hbm.at[pl.ds(NS * ZR, tail)],
                            acc.at[pl.ds(NS * ZR, tail)])
        plsc.subcore_barrier()

        for u_hbm, (row0, nrows) in zip((u0_hbm, u1_hbm, u2_hbm, u3_hbm),
                                        SLICES):
            rows_base = nrows // NW
            rows_extra = nrows % NW
            base = rows_base * wid + jnp.minimum(wid, rows_extra)

            def body(i, _, u_hbm=u_hbm, base=base, row0=row0):
                row = base + 2 * i
                grow = row0 + row
                pltpu.sync_copy(d_hbm.at[pl.ds(grow * CH, CH)], idx0)
                pltpu.sync_copy(d_hbm.at[pl.ds((grow + 1) * CH, CH)], idx1)
                pltpu.sync_copy(u_hbm.at[pl.ds(row, 2)], upd)
                a0 = pltpu.async_copy(upd.at[0], acc.at[idx0], sem, add=True)
                a1 = pltpu.async_copy(upd.at[1], acc.at[idx1], sem, add=True)
                a0.wait(); a1.wait()
                return 0

            lax.fori_loop(0, rows_base // 2, body, 0)

            if rows_extra:
                @pl.when(wid < rows_extra)
                def _(u_hbm=u_hbm, base=base, row0=row0,
                      rows_base=rows_base):
                    row = base + rows_base
                    grow = row0 + row
                    pltpu.sync_copy(d_hbm.at[pl.ds(grow * CH, CH)], idx0)
                    pltpu.sync_copy(u_hbm.at[pl.ds(row, 1)],
                                    upd.at[pl.ds(0, 1)])
                    pltpu.sync_copy(upd.at[0], acc.at[idx0], add=True)

        plsc.subcore_barrier()
        pltpu.sync_copy(acc.at[pl.ds(s * ZR, ZR)], out_hbm.at[c, pl.ds(s * ZR, ZR)])
        @pl.when(s == NS - 1)
        def _():
            tail = NN - NS * ZR
            pltpu.sync_copy(acc.at[pl.ds(NS * ZR, tail)],
                            out_hbm.at[c, pl.ds(NS * ZR, tail)])

    return scatter_k


_sc_gathers = tuple(_make_sc_gather(DN, r0, nr) for r0, nr in SLICES)
_sc_scatter = _make_sc_scatter()


def _dot0(a, b):
    return lax.dot_general(a, b, (((0,), (0,)), ((), ())),
                           preferred_element_type=jnp.float32)


def _node_msg_body(gs, gd, eft, snwt, w1s, w1e, w1d, b1, w2, b2,
                   g1s, g1e, g1d, c1, g2, c2, ww, bw, out):
    f32 = jnp.float32
    xs = gs[...]
    xd = gd[...]
    et = eft[...]
    pre_h = (jnp.dot(xs, w1s[...], preferred_element_type=f32)
             + _dot0(et, w1e[...])
             + jnp.dot(xd, w1d[...], preferred_element_type=f32) + b1[...])
    pre_g = (jnp.dot(xs, g1s[...], preferred_element_type=f32)
             + _dot0(et, g1e[...])
             + jnp.dot(xd, g1d[...], preferred_element_type=f32) + c1[...])
    h2 = _silu(jnp.dot(_silu(pre_h), w2[...], preferred_element_type=f32) + b2[...])
    gg = jax.nn.sigmoid(jnp.dot(_silu(pre_g), g2[...], preferred_element_type=f32) + c2[...])
    w = jax.nn.sigmoid(_dot0(snwt[...], ww[...]) + bw[...])
    out[...] = h2 * gg * w


def _tc_node_msg(edge0, n_edges, gs, gd, eft, snwt, w1s, w1e, w1d, b1, w2, b2,
                 g1s, g1e, g1d, c1, g2, c2, ww, bw):
    BN = 1280
    grid = n_edges // BN
    blk0 = edge0 // BN
    row = lambda i: (i, 0)
    col = lambda i: (0, i + blk0)
    full = lambda i: (0, 0)

    def wspec(a):
        return pl.BlockSpec(a.shape, full)

    return pl.pallas_call(
        _node_msg_body,
        grid=(grid,),
        in_specs=[
            pl.BlockSpec((BN, DN), row),
            pl.BlockSpec((BN, DN), row),
            pl.BlockSpec((DE, BN), col),
            pl.BlockSpec((DW, BN), col),
            wspec(w1s), wspec(w1e), wspec(w1d), wspec(b1),
            wspec(w2), wspec(b2),
            wspec(g1s), wspec(g1e), wspec(g1d), wspec(c1),
            wspec(g2), wspec(c2), wspec(ww), wspec(bw),
        ],
        out_specs=pl.BlockSpec((BN, DN), row),
        out_shape=jax.ShapeDtypeStruct((n_edges, DN), jnp.float32),
        compiler_params=pltpu.CompilerParams(
            dimension_semantics=("arbitrary",)),
    )(gs, gd, eft, snwt, w1s, w1e, w1d, b1, w2, b2,
      g1s, g1e, g1d, c1, g2, c2, ww, bw)


def _node_out_body(nf, a0, a1, ew1s, eg1s, ew1d, eg1d, nn_out, te_out):
    f32 = jnp.float32
    nn = nf[...] + a0[...] + a1[...]
    nn_out[...] = nn
    te_out[...] = jnp.concatenate(
        [jnp.dot(nn, ew1s[...], preferred_element_type=f32),
         jnp.dot(nn, eg1s[...], preferred_element_type=f32),
         jnp.dot(nn, ew1d[...], preferred_element_type=f32),
         jnp.dot(nn, eg1d[...], preferred_element_type=f32)], axis=1)


def _tc_node_out(nf, a0, a1, ew1s, eg1s, ew1d, eg1d):
    BR = 1000
    grid = NN // BR
    row = lambda i: (i, 0)
    full = lambda i: (0, 0)

    def wspec(a):
        return pl.BlockSpec(a.shape, full)

    return pl.pallas_call(
        _node_out_body,
        grid=(grid,),
        in_specs=[
            pl.BlockSpec((BR, DN), row),
            pl.BlockSpec((BR, DN), row),
            pl.BlockSpec((BR, DN), row),
            wspec(ew1s), wspec(eg1s), wspec(ew1d), wspec(eg1d),
        ],
        out_specs=[
            pl.BlockSpec((BR, DN), row),
            pl.BlockSpec((BR, DN), row),
        ],
        out_shape=[
            jax.ShapeDtypeStruct((NN, DN), jnp.float32),
            jax.ShapeDtypeStruct((NN, DN), jnp.float32),
        ],
        compiler_params=pltpu.CompilerParams(
            dimension_semantics=("arbitrary",)),
    )(nf, a0, a1, ew1s, eg1s, ew1d, eg1d)


def _edge_out_body(ges, ged, eft, sewt, ew1e, eg1e, eb1, eg1, ew2, eb2,
                   eg2, eg2b, eww, ebw, out):
    f32 = jnp.float32
    et = eft[...]
    gs = ges[...]
    gd = ged[...]
    pre_h = (gs[:, :32] + gd[:, 64:96] + _dot0(et, ew1e[...]) + eb1[...])
    pre_g = (gs[:, 32:64] + gd[:, 96:128] + _dot0(et, eg1e[...]) + eg1[...])
    h2 = _silu(jnp.dot(_silu(pre_h), ew2[...], preferred_element_type=f32) + eb2[...])
    g2 = jax.nn.sigmoid(jnp.dot(_silu(pre_g), eg2[...], preferred_element_type=f32) + eg2b[...])
    ew = jax.nn.sigmoid(_dot0(sewt[...], eww[...]) + ebw[...])
    msg = h2 * g2 * ew
    out[...] = et + jnp.swapaxes(msg, 0, 1)


def _tc_edge_out(edge0, n_edges, ges, ged, eft, sewt, ew1e, eg1e, eb1, eg1,
                 ew2, eb2, eg2, eg2b, eww, ebw):
    BN = 2560
    grid = n_edges // BN
    blk0 = edge0 // BN
    row = lambda i: (i, 0)
    col = lambda i: (0, i + blk0)
    ocol = lambda i: (0, i)
    full = lambda i: (0, 0)

    def wspec(a):
        return pl.BlockSpec(a.shape, full)

    return pl.pallas_call(
        _edge_out_body,
        grid=(grid,),
        in_specs=[
            pl.BlockSpec((BN, DN), row),
            pl.BlockSpec((BN, DN), row),
            pl.BlockSpec((DE, BN), col),
            pl.BlockSpec((DW, BN), col),
            wspec(ew1e), wspec(eg1e), wspec(eb1), wspec(eg1),
            wspec(ew2), wspec(eb2), wspec(eg2), wspec(eg2b),
            wspec(eww), wspec(ebw),
        ],
        out_specs=pl.BlockSpec((DE, BN), ocol),
        out_shape=jax.ShapeDtypeStruct((DE, n_edges), jnp.float32),
        compiler_params=pltpu.CompilerParams(
            dimension_semantics=("arbitrary",)),
    )(ges, ged, eft, sewt, ew1e, eg1e, eb1, eg1, ew2, eb2, eg2, eg2b, eww, ebw)


def kernel(node_features, edge_features, edge_index, shared_node_weights,
           shared_edge_weights, nW1, nb1, nW2, nb2, nG1, ng1, nG2, ng2,
           nWw, nbw, eW1, eb1, eW2, eb2, eG1, eg1, eG2, eg2, eWw, ebw):
    f32 = jnp.float32
    src = edge_index[0]
    dst = edge_index[1]
    eft = jnp.swapaxes(edge_features, 0, 1)
    snwt = jnp.swapaxes(shared_node_weights, 0, 1)
    sewt = jnp.swapaxes(shared_edge_weights, 0, 1)

    contribs = []
    for k, (r0, nr) in enumerate(SLICES):
        gs3, gd3 = _sc_gathers[k](node_features, src, dst)
        contrib_k = _tc_node_msg(
            r0 * CH, nr * CH,
            jnp.reshape(gs3, (nr * CH, DN)), jnp.reshape(gd3, (nr * CH, DN)),
            eft, snwt,
            nW1[:DN], nW1[DN:DN + DE], nW1[DN + DE:], nb1.reshape(1, DN),
            nW2, nb2.reshape(1, DN),
            nG1[:DN], nG1[DN:DN + DE], nG1[DN + DE:], ng1.reshape(1, DN),
            nG2, ng2.reshape(1, DN), nWw, nbw.reshape(1, DN))
        contribs.append(jnp.reshape(contrib_k, (nr, CH, DN)))

    zeros = jnp.zeros((NN, DN), f32)
    aggp = _sc_scatter(contribs[0], contribs[1], contribs[2], contribs[3],
                       dst, zeros)

    nn, te = _tc_node_out(node_features, aggp[0], aggp[1],
                          eW1[:DN], eG1[:DN], eW1[DN + DE:], eG1[DN + DE:])

    ne_parts = []
    for k, (r0, nr) in enumerate(SLICES):
        ges3, ged3 = _sc_gathers[k](te, src, dst)
        ne_parts.append(_tc_edge_out(
            r0 * CH, nr * CH,
            jnp.reshape(ges3, (nr * CH, DN)), jnp.reshape(ged3, (nr * CH, DN)),
            eft, sewt,
            eW1[DN:DN + DE], eG1[DN:DN + DE], eb1.reshape(1, 32),
            eg1.reshape(1, 32), eW2, eb2.reshape(1, DE), eG2,
            eg2.reshape(1, DE), eWw, ebw.reshape(1, DE)))

    new_edges_t = jnp.concatenate(ne_parts, axis=1)
    return nn, jnp.swapaxes(new_edges_t, 0, 1)


# speedup vs baseline: 3.7243x; 1.0681x over previous
import functools

import jax
import jax.numpy as jnp
from jax import lax
from jax.experimental import pallas as pl
from jax.experimental.pallas import tpu as pltpu
from jax.experimental.pallas import tpu_sc as plsc

NN = 10000
NE = 320000
DN = 128
DE = 16
DW = 32

NC = 2
NS = 16
NW = NC * NS
CH = 128
NROW = NE // CH
ROWS_BASE = NROW // NW
ROWS_EXTRA = NROW % NW


def _silu(x):
    return x * jax.nn.sigmoid(x)


SLICES = ((0, 640), (640, 640), (1280, 640), (1920, 580))


def _make_sc_gather(width, row0, nrows):
    mesh = plsc.VectorSubcoreMesh(core_axis_name="c", subcore_axis_name="s")
    rows_base = nrows // NW
    rows_extra = nrows % NW
    assert rows_base % 2 == 0

    @functools.partial(
        pl.kernel,
        mesh=mesh,
        out_type=(
            jax.ShapeDtypeStruct((nrows, CH, width), jnp.float32),
            jax.ShapeDtypeStruct((nrows, CH, width), jnp.float32),
        ),
        scratch_types=[
            pltpu.VMEM((2 * CH,), jnp.int32),
            pltpu.VMEM((2 * CH,), jnp.int32),
            pltpu.VMEM((2, CH, width), jnp.float32),
            pltpu.VMEM((2, CH, width), jnp.float32),
            pltpu.SemaphoreType.DMA,
            pltpu.SemaphoreType.DMA,
        ],
    )
    def gather_k(tab_hbm, s_hbm, d_hbm, gs_hbm, gd_hbm,
                 idxs, idxd, rs, rd, gsem, wsem):
        c = lax.axis_index("c")
        s = lax.axis_index("s")
        wid = s * NC + c
        base = rows_base * wid + jnp.minimum(wid, rows_extra)

        def do_pair(row, first):
            grow = row0 + row
            pltpu.sync_copy(s_hbm.at[pl.ds(grow * CH, 2 * CH)], idxs)
            pltpu.sync_copy(d_hbm.at[pl.ds(grow * CH, 2 * CH)], idxd)
            @pl.when(jnp.logical_not(first))
            def _():
                pltpu.make_async_copy(rs, gs_hbm.at[pl.ds(row, 2)], wsem).wait()
                pltpu.make_async_copy(rd, gd_hbm.at[pl.ds(row, 2)], wsem).wait()
            d0 = pltpu.async_copy(tab_hbm.at[idxs.at[pl.ds(0, CH)]], rs.at[0], gsem)
            d1 = pltpu.async_copy(tab_hbm.at[idxs.at[pl.ds(CH, CH)]], rs.at[1], gsem)
            d2 = pltpu.async_copy(tab_hbm.at[idxd.at[pl.ds(0, CH)]], rd.at[0], gsem)
            d3 = pltpu.async_copy(tab_hbm.at[idxd.at[pl.ds(CH, CH)]], rd.at[1], gsem)
            d0.wait(); d1.wait(); d2.wait(); d3.wait()
            pltpu.async_copy(rs, gs_hbm.at[pl.ds(row, 2)], wsem)
            pltpu.async_copy(rd, gd_hbm.at[pl.ds(row, 2)], wsem)

        def body(i, _):
            do_pair(base + 2 * i, i == 0)
            return 0

        lax.fori_loop(0, rows_base // 2, body, 0)
        pltpu.make_async_copy(rs, gs_hbm.at[pl.ds(base, 2)], wsem).wait()
        pltpu.make_async_copy(rd, gd_hbm.at[pl.ds(base, 2)], wsem).wait()

        if rows_extra:
            @pl.when(wid < rows_extra)
            def _():
                row = base + rows_base
                grow = row0 + row
                pltpu.sync_copy(s_hbm.at[pl.ds(grow * CH, CH)],
                                idxs.at[pl.ds(0, CH)])
                pltpu.sync_copy(d_hbm.at[pl.ds(grow * CH, CH)],
                                idxd.at[pl.ds(0, CH)])
                d0 = pltpu.async_copy(tab_hbm.at[idxs.at[pl.ds(0, CH)]],
                                      rs.at[0], gsem)
                d1 = pltpu.async_copy(tab_hbm.at[idxd.at[pl.ds(0, CH)]],
                                      rd.at[0], gsem)
                d0.wait(); d1.wait()
                pltpu.sync_copy(rs.at[pl.ds(0, 1)], gs_hbm.at[pl.ds(row, 1)])
                pltpu.sync_copy(rd.at[pl.ds(0, 1)], gd_hbm.at[pl.ds(row, 1)])

    return gather_k


def _make_sc_scatter(slice_ids):
    mesh = plsc.VectorSubcoreMesh(core_axis_name="c", subcore_axis_name="s")
    ZR = 624
    my_slices = [SLICES[i] for i in slice_ids]

    @functools.partial(
        pl.kernel,
        mesh=mesh,
        out_type=jax.ShapeDtypeStruct((NC, NN, DN), jnp.float32),
        scratch_types=[
            pltpu.VMEM((CH,), jnp.int32),
            pltpu.VMEM((CH,), jnp.int32),
            pltpu.VMEM((2, CH, DN), jnp.float32),
            pltpu.VMEM_SHARED((NN, DN), jnp.float32),
            pltpu.SemaphoreType.DMA,
        ],
    )
    def scatter_k(u0_hbm, u1_hbm, d_hbm, init_hbm, out_hbm,
                  idx0, idx1, upd, acc, sem):
        c = lax.axis_index("c")
        s = lax.axis_index("s")
        wid = s * NC + c

        pltpu.sync_copy(init_hbm.at[c, pl.ds(s * ZR, ZR)],
                        acc.at[pl.ds(s * ZR, ZR)])
        @pl.when(s == NS - 1)
        def _():
            tail = NN - NS * ZR
            pltpu.sync_copy(init_hbm.at[c, pl.ds(NS * ZR, tail)],
                            acc.at[pl.ds(NS * ZR, tail)])
        plsc.subcore_barrier()

        for u_hbm, (row0, nrows) in zip((u0_hbm, u1_hbm), my_slices):
            rows_base = nrows // NW
            rows_extra = nrows % NW
            base = rows_base * wid + jnp.minimum(wid, rows_extra)

            def body(i, _, u_hbm=u_hbm, base=base, row0=row0):
                row = base + 2 * i
                grow = row0 + row
                pltpu.sync_copy(d_hbm.at[pl.ds(grow * CH, CH)], idx0)
                pltpu.sync_copy(d_hbm.at[pl.ds((grow + 1) * CH, CH)], idx1)
                pltpu.sync_copy(u_hbm.at[pl.ds(row, 2)], upd)
                a0 = pltpu.async_copy(upd.at[0], acc.at[idx0], sem, add=True)
                a1 = pltpu.async_copy(upd.at[1], acc.at[idx1], sem, add=True)
                a0.wait(); a1.wait()
                return 0

            lax.fori_loop(0, rows_base // 2, body, 0)

            if rows_extra:
                @pl.when(wid < rows_extra)
                def _(u_hbm=u_hbm, base=base, row0=row0,
                      rows_base=rows_base):
                    row = base + rows_base
                    grow = row0 + row
                    pltpu.sync_copy(d_hbm.at[pl.ds(grow * CH, CH)], idx0)
                    pltpu.sync_copy(u_hbm.at[pl.ds(row, 1)],
                                    upd.at[pl.ds(0, 1)])
                    pltpu.sync_copy(upd.at[0], acc.at[idx0], add=True)

        plsc.subcore_barrier()
        pltpu.sync_copy(acc.at[pl.ds(s * ZR, ZR)], out_hbm.at[c, pl.ds(s * ZR, ZR)])
        @pl.when(s == NS - 1)
        def _():
            tail = NN - NS * ZR
            pltpu.sync_copy(acc.at[pl.ds(NS * ZR, tail)],
                            out_hbm.at[c, pl.ds(NS * ZR, tail)])

    return scatter_k


_sc_gathers = tuple(_make_sc_gather(DN, r0, nr) for r0, nr in SLICES)
_sc_scatter01 = _make_sc_scatter((0, 1))
_sc_scatter23 = _make_sc_scatter((2, 3))


def _dot0(a, b):
    return lax.dot_general(a, b, (((0,), (0,)), ((), ())),
                           preferred_element_type=jnp.float32)


def _node_msg_body(gs, gd, eft, snwt, w1s, w1e, w1d, b1, w2, b2,
                   g1s, g1e, g1d, c1, g2, c2, ww, bw, out):
    f32 = jnp.float32
    xs = gs[...]
    xd = gd[...]
    et = eft[...]
    pre_h = (jnp.dot(xs, w1s[...], preferred_element_type=f32)
             + _dot0(et, w1e[...])
             + jnp.dot(xd, w1d[...], preferred_element_type=f32) + b1[...])
    pre_g = (jnp.dot(xs, g1s[...], preferred_element_type=f32)
             + _dot0(et, g1e[...])
             + jnp.dot(xd, g1d[...], preferred_element_type=f32) + c1[...])
    h2 = _silu(jnp.dot(_silu(pre_h), w2[...], preferred_element_type=f32) + b2[...])
    gg = jax.nn.sigmoid(jnp.dot(_silu(pre_g), g2[...], preferred_element_type=f32) + c2[...])
    w = jax.nn.sigmoid(_dot0(snwt[...], ww[...]) + bw[...])
    out[...] = h2 * gg * w


def _tc_node_msg(edge0, n_edges, gs, gd, eft, snwt, w1s, w1e, w1d, b1, w2, b2,
                 g1s, g1e, g1d, c1, g2, c2, ww, bw):
    BN = 1280
    grid = n_edges // BN
    blk0 = edge0 // BN
    row = lambda i: (i, 0)
    col = lambda i: (0, i + blk0)
    full = lambda i: (0, 0)

    def wspec(a):
        return pl.BlockSpec(a.shape, full)

    return pl.pallas_call(
        _node_msg_body,
        grid=(grid,),
        in_specs=[
            pl.BlockSpec((BN, DN), row),
            pl.BlockSpec((BN, DN), row),
            pl.BlockSpec((DE, BN), col),
            pl.BlockSpec((DW, BN), col),
            wspec(w1s), wspec(w1e), wspec(w1d), wspec(b1),
            wspec(w2), wspec(b2),
            wspec(g1s), wspec(g1e), wspec(g1d), wspec(c1),
            wspec(g2), wspec(c2), wspec(ww), wspec(bw),
        ],
        out_specs=pl.BlockSpec((BN, DN), row),
        out_shape=jax.ShapeDtypeStruct((n_edges, DN), jnp.float32),
        compiler_params=pltpu.CompilerParams(
            dimension_semantics=("arbitrary",)),
    )(gs, gd, eft, snwt, w1s, w1e, w1d, b1, w2, b2,
      g1s, g1e, g1d, c1, g2, c2, ww, bw)


def _node_out_body(nf, a0, a1, ew1s, eg1s, ew1d, eg1d, nn_out, te_out):
    f32 = jnp.float32
    nn = nf[...] + a0[...] + a1[...]
    nn_out[...] = nn
    te_out[...] = jnp.concatenate(
        [jnp.dot(nn, ew1s[...], preferred_element_type=f32),
         jnp.dot(nn, eg1s[...], preferred_element_type=f32),
         jnp.dot(nn, ew1d[...], preferred_element_type=f32),
         jnp.dot(nn, eg1d[...], preferred_element_type=f32)], axis=1)


def _tc_node_out(nf, a0, a1, ew1s, eg1s, ew1d, eg1d):
    BR = 1000
    grid = NN // BR
    row = lambda i: (i, 0)
    full = lambda i: (0, 0)

    def wspec(a):
        return pl.BlockSpec(a.shape, full)

    return pl.pallas_call(
        _node_out_body,
        grid=(grid,),
        in_specs=[
            pl.BlockSpec((BR, DN), row),
            pl.BlockSpec((BR, DN), row),
            pl.BlockSpec((BR, DN), row),
            wspec(ew1s), wspec(eg1s), wspec(ew1d), wspec(eg1d),
        ],
        out_specs=[
            pl.BlockSpec((BR, DN), row),
            pl.BlockSpec((BR, DN), row),
        ],
        out_shape=[
            jax.ShapeDtypeStruct((NN, DN), jnp.float32),
            jax.ShapeDtypeStruct((NN, DN), jnp.float32),
        ],
        compiler_params=pltpu.CompilerParams(
            dimension_semantics=("arbitrary",)),
    )(nf, a0, a1, ew1s, eg1s, ew1d, eg1d)


def _edge_out_body(ges, ged, eft, sewt, ew1e, eg1e, eb1, eg1, ew2, eb2,
                   eg2, eg2b, eww, ebw, out):
    f32 = jnp.float32
    et = eft[...]
    gs = ges[...]
    gd = ged[...]
    pre_h = (gs[:, :32] + gd[:, 64:96] + _dot0(et, ew1e[...]) + eb1[...])
    pre_g = (gs[:, 32:64] + gd[:, 96:128] + _dot0(et, eg1e[...]) + eg1[...])
    h2 = _silu(jnp.dot(_silu(pre_h), ew2[...], preferred_element_type=f32) + eb2[...])
    g2 = jax.nn.sigmoid(jnp.dot(_silu(pre_g), eg2[...], preferred_element_type=f32) + eg2b[...])
    ew = jax.nn.sigmoid(_dot0(sewt[...], eww[...]) + ebw[...])
    msg = h2 * g2 * ew
    out[...] = et + jnp.swapaxes(msg, 0, 1)


def _tc_edge_out(edge0, n_edges, ges, ged, eft, sewt, ew1e, eg1e, eb1, eg1,
                 ew2, eb2, eg2, eg2b, eww, ebw):
    BN = 2560
    grid = n_edges // BN
    blk0 = edge0 // BN
    row = lambda i: (i, 0)
    col = lambda i: (0, i + blk0)
    ocol = lambda i: (0, i)
    full = lambda i: (0, 0)

    def wspec(a):
        return pl.BlockSpec(a.shape, full)

    return pl.pallas_call(
        _edge_out_body,
        grid=(grid,),
        in_specs=[
            pl.BlockSpec((BN, DN), row),
            pl.BlockSpec((BN, DN), row),
            pl.BlockSpec((DE, BN), col),
            pl.BlockSpec((DW, BN), col),
            wspec(ew1e), wspec(eg1e), wspec(eb1), wspec(eg1),
            wspec(ew2), wspec(eb2), wspec(eg2), wspec(eg2b),
            wspec(eww), wspec(ebw),
        ],
        out_specs=pl.BlockSpec((DE, BN), ocol),
        out_shape=jax.ShapeDtypeStruct((DE, n_edges), jnp.float32),
        compiler_params=pltpu.CompilerParams(
            dimension_semantics=("arbitrary",)),
    )(ges, ged, eft, sewt, ew1e, eg1e, eb1, eg1, ew2, eb2, eg2, eg2b, eww, ebw)


def kernel(node_features, edge_features, edge_index, shared_node_weights,
           shared_edge_weights, nW1, nb1, nW2, nb2, nG1, ng1, nG2, ng2,
           nWw, nbw, eW1, eb1, eW2, eb2, eG1, eg1, eG2, eg2, eWw, ebw):
    f32 = jnp.float32
    src = edge_index[0]
    dst = edge_index[1]
    eft = jnp.swapaxes(edge_features, 0, 1)
    snwt = jnp.swapaxes(shared_node_weights, 0, 1)
    sewt = jnp.swapaxes(shared_edge_weights, 0, 1)

    contribs = []
    for k, (r0, nr) in enumerate(SLICES):
        gs3, gd3 = _sc_gathers[k](node_features, src, dst)
        contrib_k = _tc_node_msg(
            r0 * CH, nr * CH,
            jnp.reshape(gs3, (nr * CH, DN)), jnp.reshape(gd3, (nr * CH, DN)),
            eft, snwt,
            nW1[:DN], nW1[DN:DN + DE], nW1[DN + DE:], nb1.reshape(1, DN),
            nW2, nb2.reshape(1, DN),
            nG1[:DN], nG1[DN:DN + DE], nG1[DN + DE:], ng1.reshape(1, DN),
            nG2, ng2.reshape(1, DN), nWw, nbw.reshape(1, DN))
        contribs.append(jnp.reshape(contrib_k, (nr, CH, DN)))

    zeros = jnp.zeros((NC, NN, DN), f32)
    agg01 = _sc_scatter01(contribs[0], contribs[1], dst, zeros)
    aggp = _sc_scatter23(contribs[2], contribs[3], dst, agg01)

    nn, te = _tc_node_out(node_features, aggp[0], aggp[1],
                          eW1[:DN], eG1[:DN], eW1[DN + DE:], eG1[DN + DE:])

    ne_parts = []
    for k, (r0, nr) in enumerate(SLICES):
        ges3, ged3 = _sc_gathers[k](te, src, dst)
        ne_parts.append(_tc_edge_out(
            r0 * CH, nr * CH,
            jnp.reshape(ges3, (nr * CH, DN)), jnp.reshape(ged3, (nr * CH, DN)),
            eft, sewt,
            eW1[DN:DN + DE], eG1[DN:DN + DE], eb1.reshape(1, 32),
            eg1.reshape(1, 32), eW2, eb2.reshape(1, DE), eG2,
            eg2.reshape(1, DE), eWw, ebw.reshape(1, DE)))

    new_edges_t = jnp.concatenate(ne_parts, axis=1)
    return nn, jnp.swapaxes(new_edges_t, 0, 1)


# speedup vs baseline: 3.9616x; 1.0637x over previous
import functools

import jax
import jax.numpy as jnp
from jax import lax
from jax.experimental import pallas as pl
from jax.experimental.pallas import tpu as pltpu
from jax.experimental.pallas import tpu_sc as plsc

NN = 10000
NE = 320000
DN = 128
DE = 16
DW = 32

NC = 2
NS = 16
NW = NC * NS
CH = 128
NROW = NE // CH
ROWS_BASE = NROW // NW
ROWS_EXTRA = NROW % NW


def _silu(x):
    return x * jax.nn.sigmoid(x)


SLICES = ((0, 640), (640, 640), (1280, 640), (1920, 580))


def _make_sc_gather(width, row0, nrows):
    mesh = plsc.VectorSubcoreMesh(core_axis_name="c", subcore_axis_name="s")
    rows_base = nrows // NW
    rows_extra = nrows % NW
    assert rows_base % 2 == 0

    @functools.partial(
        pl.kernel,
        mesh=mesh,
        out_type=(
            jax.ShapeDtypeStruct((nrows, CH, width), jnp.float32),
            jax.ShapeDtypeStruct((nrows, CH, width), jnp.float32),
        ),
        scratch_types=[
            pltpu.VMEM((2 * CH,), jnp.int32),
            pltpu.VMEM((2 * CH,), jnp.int32),
            pltpu.VMEM((2, CH, width), jnp.float32),
            pltpu.VMEM((2, CH, width), jnp.float32),
            pltpu.SemaphoreType.DMA,
            pltpu.SemaphoreType.DMA,
        ],
    )
    def gather_k(tab_hbm, s_hbm, d_hbm, gs_hbm, gd_hbm,
                 idxs, idxd, rs, rd, gsem, wsem):
        c = lax.axis_index("c")
        s = lax.axis_index("s")
        wid = s * NC + c
        base = rows_base * wid + jnp.minimum(wid, rows_extra)

        def do_pair(row, first):
            grow = row0 + row
            pltpu.sync_copy(s_hbm.at[pl.ds(grow * CH, 2 * CH)], idxs)
            pltpu.sync_copy(d_hbm.at[pl.ds(grow * CH, 2 * CH)], idxd)
            @pl.when(jnp.logical_not(first))
            def _():
                pltpu.make_async_copy(rs, gs_hbm.at[pl.ds(row, 2)], wsem).wait()
                pltpu.make_async_copy(rd, gd_hbm.at[pl.ds(row, 2)], wsem).wait()
            d0 = pltpu.async_copy(tab_hbm.at[idxs.at[pl.ds(0, CH)]], rs.at[0], gsem)
            d1 = pltpu.async_copy(tab_hbm.at[idxs.at[pl.ds(CH, CH)]], rs.at[1], gsem)
            d2 = pltpu.async_copy(tab_hbm.at[idxd.at[pl.ds(0, CH)]], rd.at[0], gsem)
            d3 = pltpu.async_copy(tab_hbm.at[idxd.at[pl.ds(CH, CH)]], rd.at[1], gsem)
            d0.wait(); d1.wait(); d2.wait(); d3.wait()
            pltpu.async_copy(rs, gs_hbm.at[pl.ds(row, 2)], wsem)
            pltpu.async_copy(rd, gd_hbm.at[pl.ds(row, 2)], wsem)

        def body(i, _):
            do_pair(base + 2 * i, i == 0)
            return 0

        lax.fori_loop(0, rows_base // 2, body, 0)
        pltpu.make_async_copy(rs, gs_hbm.at[pl.ds(base, 2)], wsem).wait()
        pltpu.make_async_copy(rd, gd_hbm.at[pl.ds(base, 2)], wsem).wait()

        if rows_extra:
            @pl.when(wid < rows_extra)
            def _():
                row = base + rows_base
                grow = row0 + row
                pltpu.sync_copy(s_hbm.at[pl.ds(grow * CH, CH)],
                                idxs.at[pl.ds(0, CH)])
                pltpu.sync_copy(d_hbm.at[pl.ds(grow * CH, CH)],
                                idxd.at[pl.ds(0, CH)])
                d0 = pltpu.async_copy(tab_hbm.at[idxs.at[pl.ds(0, CH)]],
                                      rs.at[0], gsem)
                d1 = pltpu.async_copy(tab_hbm.at[idxd.at[pl.ds(0, CH)]],
                                      rd.at[0], gsem)
                d0.wait(); d1.wait()
                pltpu.sync_copy(rs.at[pl.ds(0, 1)], gs_hbm.at[pl.ds(row, 1)])
                pltpu.sync_copy(rd.at[pl.ds(0, 1)], gd_hbm.at[pl.ds(row, 1)])

    return gather_k


def _make_sc_gather_pack(row0, nrows):
    W = DN
    H = W // 2
    mesh = plsc.VectorSubcoreMesh(core_axis_name="c", subcore_axis_name="s")
    ch_base = nrows // NW
    ch_extra = nrows % NW
    assert ch_base % 2 == 0

    @functools.partial(
        pl.kernel,
        mesh=mesh,
        out_type=jax.ShapeDtypeStruct((nrows, CH, W), jnp.float32),
        scratch_types=[
            pltpu.VMEM((CH,), jnp.int32),
            pltpu.VMEM((CH,), jnp.int32),
            pltpu.VMEM((CH,), jnp.int32),
            pltpu.VMEM((CH,), jnp.int32),
            pltpu.VMEM((CH, W), jnp.float32),
            pltpu.VMEM((CH, W), jnp.float32),
            pltpu.VMEM((CH, W), jnp.float32),
            pltpu.VMEM((CH, W), jnp.float32),
            pltpu.VMEM((1, CH, W), jnp.float32),
            pltpu.VMEM((1, CH, W), jnp.float32),
            pltpu.SemaphoreType.DMA,
            pltpu.SemaphoreType.DMA,
        ],
    )
    def gather_k(tab_hbm, s_hbm, d_hbm, g_hbm,
                 is0, is1, id0, id1, rs0, rs1, rd0, rd1, pk0, pk1,
                 gsem, wsem):
        c = lax.axis_index("c")
        s = lax.axis_index("s")
        wid = s * NC + c
        base = ch_base * wid + jnp.minimum(wid, ch_extra)
        isb = (is0, is1)
        idb = (id0, id1)
        rsb = (rs0, rs1)
        rdb = (rd0, rd1)
        pkb = (pk0, pk1)

        def fire(row, p):
            grow = row0 + row
            pltpu.sync_copy(s_hbm.at[pl.ds(grow * CH, CH)], isb[p])
            pltpu.sync_copy(d_hbm.at[pl.ds(grow * CH, CH)], idb[p])
            pltpu.async_copy(tab_hbm.at[isb[p]], rsb[p], gsem)
            pltpu.async_copy(tab_hbm.at[idb[p]], rdb[p], gsem)

        def merge_wb(row, p, first):
            pltpu.make_async_copy(tab_hbm.at[isb[p]], rsb[p], gsem).wait()
            pltpu.make_async_copy(tab_hbm.at[idb[p]], rdb[p], gsem).wait()
            @pl.when(jnp.logical_not(first))
            def _():
                pltpu.make_async_copy(pkb[p], g_hbm.at[pl.ds(row, 1)],
                                      wsem).wait()
            rs, rd, pk = rsb[p], rdb[p], pkb[p]

            def mrow(r, _):
                for k in range(H // 16):
                    pk[0, r, pl.ds(k * 16, 16)] = rs[r, pl.ds(k * 16, 16)]
                    pk[0, r, pl.ds(H + k * 16, 16)] = rd[r, pl.ds(H + k * 16, 16)]
                return 0

            lax.fori_loop(0, CH, mrow, 0)
            pltpu.async_copy(pk, g_hbm.at[pl.ds(row, 1)], wsem)

        fire(base, 0)

        def body(i, _):
            r0 = base + 2 * i
            fire(r0 + 1, 1)
            merge_wb(r0, 0, i == 0)
            @pl.when(2 * i + 2 < ch_base)
            def _():
                fire(r0 + 2, 0)
            merge_wb(r0 + 1, 1, i == 0)
            return 0

        lax.fori_loop(0, ch_base // 2, body, 0)
        pltpu.make_async_copy(pk0, g_hbm.at[pl.ds(base, 1)], wsem).wait()
        pltpu.make_async_copy(pk1, g_hbm.at[pl.ds(base, 1)], wsem).wait()

        if ch_extra:
            @pl.when(wid < ch_extra)
            def _():
                row = base + ch_base
                fire(row, 0)
                pltpu.make_async_copy(tab_hbm.at[is0], rs0, gsem).wait()
                pltpu.make_async_copy(tab_hbm.at[id0], rd0, gsem).wait()

                def mrow(r, _):
                    for k in range(H // 16):
                        pk0[0, r, pl.ds(k * 16, 16)] = rs0[r, pl.ds(k * 16, 16)]
                        pk0[0, r, pl.ds(H + k * 16, 16)] = rd0[r, pl.ds(H + k * 16, 16)]
                    return 0

                lax.fori_loop(0, CH, mrow, 0)
                pltpu.sync_copy(pk0, g_hbm.at[pl.ds(row, 1)])

    return gather_k


def _make_sc_scatter(slice_ids):
    mesh = plsc.VectorSubcoreMesh(core_axis_name="c", subcore_axis_name="s")
    ZR = 624
    my_slices = [SLICES[i] for i in slice_ids]

    @functools.partial(
        pl.kernel,
        mesh=mesh,
        out_type=jax.ShapeDtypeStruct((NC, NN, DN), jnp.float32),
        scratch_types=[
            pltpu.VMEM((CH,), jnp.int32),
            pltpu.VMEM((CH,), jnp.int32),
            pltpu.VMEM((2, CH, DN), jnp.float32),
            pltpu.VMEM_SHARED((NN, DN), jnp.float32),
            pltpu.SemaphoreType.DMA,
        ],
    )
    def scatter_k(u0_hbm, u1_hbm, d_hbm, init_hbm, out_hbm,
                  idx0, idx1, upd, acc, sem):
        c = lax.axis_index("c")
        s = lax.axis_index("s")
        wid = s * NC + c

        pltpu.sync_copy(init_hbm.at[c, pl.ds(s * ZR, ZR)],
                        acc.at[pl.ds(s * ZR, ZR)])
        @pl.when(s == NS - 1)
        def _():
            tail = NN - NS * ZR
            pltpu.sync_copy(init_hbm.at[c, pl.ds(NS * ZR, tail)],
                            acc.at[pl.ds(NS * ZR, tail)])
        plsc.subcore_barrier()

        for u_hbm, (row0, nrows) in zip((u0_hbm, u1_hbm), my_slices):
            rows_base = nrows // NW
            rows_extra = nrows % NW
            base = rows_base * wid + jnp.minimum(wid, rows_extra)

            def body(i, _, u_hbm=u_hbm, base=base, row0=row0):
                row = base + 2 * i
                grow = row0 + row
                pltpu.sync_copy(d_hbm.at[pl.ds(grow * CH, CH)], idx0)
                pltpu.sync_copy(d_hbm.at[pl.ds((grow + 1) * CH, CH)], idx1)
                pltpu.sync_copy(u_hbm.at[pl.ds(row, 2)], upd)
                a0 = pltpu.async_copy(upd.at[0], acc.at[idx0], sem, add=True)
                a1 = pltpu.async_copy(upd.at[1], acc.at[idx1], sem, add=True)
                a0.wait(); a1.wait()
                return 0

            lax.fori_loop(0, rows_base // 2, body, 0)

            if rows_extra:
                @pl.when(wid < rows_extra)
                def _(u_hbm=u_hbm, base=base, row0=row0,
                      rows_base=rows_base):
                    row = base + rows_base
                    grow = row0 + row
                    pltpu.sync_copy(d_hbm.at[pl.ds(grow * CH, CH)], idx0)
                    pltpu.sync_copy(u_hbm.at[pl.ds(row, 1)],
                                    upd.at[pl.ds(0, 1)])
                    pltpu.sync_copy(upd.at[0], acc.at[idx0], add=True)

        plsc.subcore_barrier()
        pltpu.sync_copy(acc.at[pl.ds(s * ZR, ZR)], out_hbm.at[c, pl.ds(s * ZR, ZR)])
        @pl.when(s == NS - 1)
        def _():
            tail = NN - NS * ZR
            pltpu.sync_copy(acc.at[pl.ds(NS * ZR, tail)],
                            out_hbm.at[c, pl.ds(NS * ZR, tail)])

    return scatter_k


_sc_gathers = tuple(_make_sc_gather(DN, r0, nr) for r0, nr in SLICES)
_sc_gathers_pack = tuple(_make_sc_gather_pack(r0, nr) for r0, nr in SLICES)
_sc_scatter01 = _make_sc_scatter((0, 1))
_sc_scatter23 = _make_sc_scatter((2, 3))


def _dot0(a, b):
    return lax.dot_general(a, b, (((0,), (0,)), ((), ())),
                           preferred_element_type=jnp.float32)


def _node_msg_body(gs, gd, eft, snwt, w1s, w1e, w1d, b1, w2, b2,
                   g1s, g1e, g1d, c1, g2, c2, ww, bw, out):
    f32 = jnp.float32
    xs = gs[...]
    xd = gd[...]
    et = eft[...]
    pre_h = (jnp.dot(xs, w1s[...], preferred_element_type=f32)
             + _dot0(et, w1e[...])
             + jnp.dot(xd, w1d[...], preferred_element_type=f32) + b1[...])
    pre_g = (jnp.dot(xs, g1s[...], preferred_element_type=f32)
             + _dot0(et, g1e[...])
             + jnp.dot(xd, g1d[...], preferred_element_type=f32) + c1[...])
    h2 = _silu(jnp.dot(_silu(pre_h), w2[...], preferred_element_type=f32) + b2[...])
    gg = jax.nn.sigmoid(jnp.dot(_silu(pre_g), g2[...], preferred_element_type=f32) + c2[...])
    w = jax.nn.sigmoid(_dot0(snwt[...], ww[...]) + bw[...])
    out[...] = h2 * gg * w


def _tc_node_msg(edge0, n_edges, gs, gd, eft, snwt, w1s, w1e, w1d, b1, w2, b2,
                 g1s, g1e, g1d, c1, g2, c2, ww, bw):
    BN = 1280
    grid = n_edges // BN
    blk0 = edge0 // BN
    row = lambda i: (i, 0)
    col = lambda i: (0, i + blk0)
    full = lambda i: (0, 0)

    def wspec(a):
        return pl.BlockSpec(a.shape, full)

    return pl.pallas_call(
        _node_msg_body,
        grid=(grid,),
        in_specs=[
            pl.BlockSpec((BN, DN), row),
            pl.BlockSpec((BN, DN), row),
            pl.BlockSpec((DE, BN), col),
            pl.BlockSpec((DW, BN), col),
            wspec(w1s), wspec(w1e), wspec(w1d), wspec(b1),
            wspec(w2), wspec(b2),
            wspec(g1s), wspec(g1e), wspec(g1d), wspec(c1),
            wspec(g2), wspec(c2), wspec(ww), wspec(bw),
        ],
        out_specs=pl.BlockSpec((BN, DN), row),
        out_shape=jax.ShapeDtypeStruct((n_edges, DN), jnp.float32),
        compiler_params=pltpu.CompilerParams(
            dimension_semantics=("arbitrary",)),
    )(gs, gd, eft, snwt, w1s, w1e, w1d, b1, w2, b2,
      g1s, g1e, g1d, c1, g2, c2, ww, bw)


def _node_out_body(nf, a0, a1, ew1s, eg1s, ew1d, eg1d, nn_out, te_out):
    f32 = jnp.float32
    nn = nf[...] + a0[...] + a1[...]
    nn_out[...] = nn
    te_out[...] = jnp.concatenate(
        [jnp.dot(nn, ew1s[...], preferred_element_type=f32),
         jnp.dot(nn, eg1s[...], preferred_element_type=f32),
         jnp.dot(nn, ew1d[...], preferred_element_type=f32),
         jnp.dot(nn, eg1d[...], preferred_element_type=f32)], axis=1)


def _tc_node_out(nf, a0, a1, ew1s, eg1s, ew1d, eg1d):
    BR = 1000
    grid = NN // BR
    row = lambda i: (i, 0)
    full = lambda i: (0, 0)

    def wspec(a):
        return pl.BlockSpec(a.shape, full)

    return pl.pallas_call(
        _node_out_body,
        grid=(grid,),
        in_specs=[
            pl.BlockSpec((BR, DN), row),
            pl.BlockSpec((BR, DN), row),
            pl.BlockSpec((BR, DN), row),
            wspec(ew1s), wspec(eg1s), wspec(ew1d), wspec(eg1d),
        ],
        out_specs=[
            pl.BlockSpec((BR, DN), row),
            pl.BlockSpec((BR, DN), row),
        ],
        out_shape=[
            jax.ShapeDtypeStruct((NN, DN), jnp.float32),
            jax.ShapeDtypeStruct((NN, DN), jnp.float32),
        ],
        compiler_params=pltpu.CompilerParams(
            dimension_semantics=("arbitrary",)),
    )(nf, a0, a1, ew1s, eg1s, ew1d, eg1d)


def _edge_out_body(ge, eft, sewt, ew1e, eg1e, eb1, eg1, ew2, eb2,
                   eg2, eg2b, eww, ebw, out):
    f32 = jnp.float32
    et = eft[...]
    g = ge[...]
    pre_h = (g[:, :32] + g[:, 64:96] + _dot0(et, ew1e[...]) + eb1[...])
    pre_g = (g[:, 32:64] + g[:, 96:128] + _dot0(et, eg1e[...]) + eg1[...])
    h2 = _silu(jnp.dot(_silu(pre_h), ew2[...], preferred_element_type=f32) + eb2[...])
    g2 = jax.nn.sigmoid(jnp.dot(_silu(pre_g), eg2[...], preferred_element_type=f32) + eg2b[...])
    ew = jax.nn.sigmoid(_dot0(sewt[...], eww[...]) + ebw[...])
    msg = h2 * g2 * ew
    out[...] = et + jnp.swapaxes(msg, 0, 1)


def _tc_edge_out(edge0, n_edges, ge, eft, sewt, ew1e, eg1e, eb1, eg1,
                 ew2, eb2, eg2, eg2b, eww, ebw):
    BN = 2560
    grid = n_edges // BN
    blk0 = edge0 // BN
    row = lambda i: (i, 0)
    col = lambda i: (0, i + blk0)
    ocol = lambda i: (0, i)
    full = lambda i: (0, 0)

    def wspec(a):
        return pl.BlockSpec(a.shape, full)

    return pl.pallas_call(
        _edge_out_body,
        grid=(grid,),
        in_specs=[
            pl.BlockSpec((BN, DN), row),
            pl.BlockSpec((DE, BN), col),
            pl.BlockSpec((DW, BN), col),
            wspec(ew1e), wspec(eg1e), wspec(eb1), wspec(eg1),
            wspec(ew2), wspec(eb2), wspec(eg2), wspec(eg2b),
            wspec(eww), wspec(ebw),
        ],
        out_specs=pl.BlockSpec((DE, BN), ocol),
        out_shape=jax.ShapeDtypeStruct((DE, n_edges), jnp.float32),
        compiler_params=pltpu.CompilerParams(
            dimension_semantics=("arbitrary",)),
    )(ge, eft, sewt, ew1e, eg1e, eb1, eg1, ew2, eb2, eg2, eg2b, eww, ebw)


def kernel(node_features, edge_features, edge_index, shared_node_weights,
           shared_edge_weights, nW1, nb1, nW2, nb2, nG1, ng1, nG2, ng2,
           nWw, nbw, eW1, eb1, eW2, eb2, eG1, eg1, eG2, eg2, eWw, ebw):
    f32 = jnp.float32
    src = edge_index[0]
    dst = edge_index[1]
    eft = jnp.swapaxes(edge_features, 0, 1)
    snwt = jnp.swapaxes(shared_node_weights, 0, 1)
    sewt = jnp.swapaxes(shared_edge_weights, 0, 1)

    contribs = []
    for k, (r0, nr) in enumerate(SLICES):
        gs3, gd3 = _sc_gathers[k](node_features, src, dst)
        contrib_k = _tc_node_msg(
            r0 * CH, nr * CH,
            jnp.reshape(gs3, (nr * CH, DN)), jnp.reshape(gd3, (nr * CH, DN)),
            eft, snwt,
            nW1[:DN], nW1[DN:DN + DE], nW1[DN + DE:], nb1.reshape(1, DN),
            nW2, nb2.reshape(1, DN),
            nG1[:DN], nG1[DN:DN + DE], nG1[DN + DE:], ng1.reshape(1, DN),
            nG2, ng2.reshape(1, DN), nWw, nbw.reshape(1, DN))
        contribs.append(jnp.reshape(contrib_k, (nr, CH, DN)))

    zeros = jnp.zeros((NC, NN, DN), f32)
    agg01 = _sc_scatter01(contribs[0], contribs[1], dst, zeros)
    aggp = _sc_scatter23(contribs[2], contribs[3], dst, agg01)

    nn, te = _tc_node_out(node_features, aggp[0], aggp[1],
                          eW1[:DN], eG1[:DN], eW1[DN + DE:], eG1[DN + DE:])

    ne_parts = []
    for k, (r0, nr) in enumerate(SLICES):
        ge3 = _sc_gathers_pack[k](te, src, dst)
        ne_parts.append(_tc_edge_out(
            r0 * CH, nr * CH,
            jnp.reshape(ge3, (nr * CH, DN)),
            eft, sewt,
            eW1[DN:DN + DE], eG1[DN:DN + DE], eb1.reshape(1, 32),
            eg1.reshape(1, 32), eW2, eb2.reshape(1, DE), eG2,
            eg2.reshape(1, DE), eWw, ebw.reshape(1, DE)))

    new_edges_t = jnp.concatenate(ne_parts, axis=1)
    return nn, jnp.swapaxes(new_edges_t, 0, 1)


# speedup vs baseline: 4.0072x; 1.0115x over previous
import functools

import jax
import jax.numpy as jnp
from jax import lax
from jax.experimental import pallas as pl
from jax.experimental.pallas import tpu as pltpu
from jax.experimental.pallas import tpu_sc as plsc

NN = 10000
NE = 320000
DN = 128
DE = 16
DW = 32

NC = 2
NS = 16
NW = NC * NS
CH = 128
NROW = NE // CH
ROWS_BASE = NROW // NW
ROWS_EXTRA = NROW % NW


def _silu(x):
    return x * jax.nn.sigmoid(x)


SLICES = ((0, 640), (640, 640), (1280, 640), (1920, 580))


def _make_sc_gather(width, row0, nrows):
    mesh = plsc.VectorSubcoreMesh(core_axis_name="c", subcore_axis_name="s")
    rows_base = nrows // NW
    rows_extra = nrows % NW
    assert rows_base % 2 == 0

    @functools.partial(
        pl.kernel,
        mesh=mesh,
        out_type=(
            jax.ShapeDtypeStruct((nrows, CH, width), jnp.float32),
            jax.ShapeDtypeStruct((nrows, CH, width), jnp.float32),
        ),
        scratch_types=[
            pltpu.VMEM((2 * CH,), jnp.int32),
            pltpu.VMEM((2 * CH,), jnp.int32),
            pltpu.VMEM((2, CH, width), jnp.float32),
            pltpu.VMEM((2, CH, width), jnp.float32),
            pltpu.SemaphoreType.DMA,
            pltpu.SemaphoreType.DMA,
        ],
    )
    def gather_k(tab_hbm, s_hbm, d_hbm, gs_hbm, gd_hbm,
                 idxs, idxd, rs, rd, gsem, wsem):
        c = lax.axis_index("c")
        s = lax.axis_index("s")
        wid = s * NC + c
        base = rows_base * wid + jnp.minimum(wid, rows_extra)

        def do_pair(row, first):
            grow = row0 + row
            pltpu.sync_copy(s_hbm.at[pl.ds(grow * CH, 2 * CH)], idxs)
            pltpu.sync_copy(d_hbm.at[pl.ds(grow * CH, 2 * CH)], idxd)
            @pl.when(jnp.logical_not(first))
            def _():
                pltpu.make_async_copy(rs, gs_hbm.at[pl.ds(row, 2)], wsem).wait()
                pltpu.make_async_copy(rd, gd_hbm.at[pl.ds(row, 2)], wsem).wait()
            d0 = pltpu.async_copy(tab_hbm.at[idxs.at[pl.ds(0, CH)]], rs.at[0], gsem)
            d1 = pltpu.async_copy(tab_hbm.at[idxs.at[pl.ds(CH, CH)]], rs.at[1], gsem)
            d2 = pltpu.async_copy(tab_hbm.at[idxd.at[pl.ds(0, CH)]], rd.at[0], gsem)
            d3 = pltpu.async_copy(tab_hbm.at[idxd.at[pl.ds(CH, CH)]], rd.at[1], gsem)
            d0.wait(); d1.wait(); d2.wait(); d3.wait()
            pltpu.async_copy(rs, gs_hbm.at[pl.ds(row, 2)], wsem)
            pltpu.async_copy(rd, gd_hbm.at[pl.ds(row, 2)], wsem)

        def body(i, _):
            do_pair(base + 2 * i, i == 0)
            return 0

        lax.fori_loop(0, rows_base // 2, body, 0)
        pltpu.make_async_copy(rs, gs_hbm.at[pl.ds(base, 2)], wsem).wait()
        pltpu.make_async_copy(rd, gd_hbm.at[pl.ds(base, 2)], wsem).wait()

        if rows_extra:
            @pl.when(wid < rows_extra)
            def _():
                row = base + rows_base
                grow = row0 + row
                pltpu.sync_copy(s_hbm.at[pl.ds(grow * CH, CH)],
                                idxs.at[pl.ds(0, CH)])
                pltpu.sync_copy(d_hbm.at[pl.ds(grow * CH, CH)],
                                idxd.at[pl.ds(0, CH)])
                d0 = pltpu.async_copy(tab_hbm.at[idxs.at[pl.ds(0, CH)]],
                                      rs.at[0], gsem)
                d1 = pltpu.async_copy(tab_hbm.at[idxd.at[pl.ds(0, CH)]],
                                      rd.at[0], gsem)
                d0.wait(); d1.wait()
                pltpu.sync_copy(rs.at[pl.ds(0, 1)], gs_hbm.at[pl.ds(row, 1)])
                pltpu.sync_copy(rd.at[pl.ds(0, 1)], gd_hbm.at[pl.ds(row, 1)])

    return gather_k


def _make_sc_gather_pack(row0, nrows):
    W = DN
    H = W // 2
    mesh = plsc.VectorSubcoreMesh(core_axis_name="c", subcore_axis_name="s")
    ch_base = nrows // NW
    ch_extra = nrows % NW
    assert ch_base % 2 == 0

    @functools.partial(
        pl.kernel,
        mesh=mesh,
        out_type=jax.ShapeDtypeStruct((nrows, CH, W), jnp.float32),
        scratch_types=[
            pltpu.VMEM((CH,), jnp.int32),
            pltpu.VMEM((CH,), jnp.int32),
            pltpu.VMEM((CH,), jnp.int32),
            pltpu.VMEM((CH,), jnp.int32),
            pltpu.VMEM((CH, W), jnp.float32),
            pltpu.VMEM((CH, W), jnp.float32),
            pltpu.VMEM((CH, W), jnp.float32),
            pltpu.VMEM((CH, W), jnp.float32),
            pltpu.VMEM((1, CH, W), jnp.float32),
            pltpu.VMEM((1, CH, W), jnp.float32),
            pltpu.SemaphoreType.DMA,
            pltpu.SemaphoreType.DMA,
        ],
    )
    def gather_k(tab_hbm, s_hbm, d_hbm, g_hbm,
                 is0, is1, id0, id1, rs0, rs1, rd0, rd1, pk0, pk1,
                 gsem, wsem):
        c = lax.axis_index("c")
        s = lax.axis_index("s")
        wid = s * NC + c
        base = ch_base * wid + jnp.minimum(wid, ch_extra)
        isb = (is0, is1)
        idb = (id0, id1)
        rsb = (rs0, rs1)
        rdb = (rd0, rd1)
        pkb = (pk0, pk1)

        def fire(row, p):
            grow = row0 + row
            pltpu.sync_copy(s_hbm.at[pl.ds(grow * CH, CH)], isb[p])
            pltpu.sync_copy(d_hbm.at[pl.ds(grow * CH, CH)], idb[p])
            pltpu.async_copy(tab_hbm.at[isb[p]], rsb[p], gsem)
            pltpu.async_copy(tab_hbm.at[idb[p]], rdb[p], gsem)

        def merge_wb(row, p, first):
            pltpu.make_async_copy(tab_hbm.at[isb[p]], rsb[p], gsem).wait()
            pltpu.make_async_copy(tab_hbm.at[idb[p]], rdb[p], gsem).wait()
            @pl.when(jnp.logical_not(first))
            def _():
                pltpu.make_async_copy(pkb[p], g_hbm.at[pl.ds(row, 1)],
                                      wsem).wait()
            rs, rd, pk = rsb[p], rdb[p], pkb[p]

            def mrow(r, _):
                for k in range(H // 16):
                    pk[0, r, pl.ds(k * 16, 16)] = rs[r, pl.ds(k * 16, 16)]
                    pk[0, r, pl.ds(H + k * 16, 16)] = rd[r, pl.ds(H + k * 16, 16)]
                return 0

            lax.fori_loop(0, CH, mrow, 0)
            pltpu.async_copy(pk, g_hbm.at[pl.ds(row, 1)], wsem)

        fire(base, 0)

        def body(i, _):
            r0 = base + 2 * i
            fire(r0 + 1, 1)
            merge_wb(r0, 0, i == 0)
            @pl.when(2 * i + 2 < ch_base)
            def _():
                fire(r0 + 2, 0)
            merge_wb(r0 + 1, 1, i == 0)
            return 0

        lax.fori_loop(0, ch_base // 2, body, 0)
        pltpu.make_async_copy(pk0, g_hbm.at[pl.ds(base, 1)], wsem).wait()
        pltpu.make_async_copy(pk1, g_hbm.at[pl.ds(base, 1)], wsem).wait()

        if ch_extra:
            @pl.when(wid < ch_extra)
            def _():
                row = base + ch_base
                fire(row, 0)
                pltpu.make_async_copy(tab_hbm.at[is0], rs0, gsem).wait()
                pltpu.make_async_copy(tab_hbm.at[id0], rd0, gsem).wait()

                def mrow(r, _):
                    for k in range(H // 16):
                        pk0[0, r, pl.ds(k * 16, 16)] = rs0[r, pl.ds(k * 16, 16)]
                        pk0[0, r, pl.ds(H + k * 16, 16)] = rd0[r, pl.ds(H + k * 16, 16)]
                    return 0

                lax.fori_loop(0, CH, mrow, 0)
                pltpu.sync_copy(pk0, g_hbm.at[pl.ds(row, 1)])

    return gather_k


def _make_sc_scatter(slice_ids):
    mesh = plsc.VectorSubcoreMesh(core_axis_name="c", subcore_axis_name="s")
    ZR = 624
    my_slices = [SLICES[i] for i in slice_ids]

    @functools.partial(
        pl.kernel,
        mesh=mesh,
        out_type=jax.ShapeDtypeStruct((NC, NN, DN), jnp.float32),
        scratch_types=[
            pltpu.VMEM((CH,), jnp.int32),
            pltpu.VMEM((CH,), jnp.int32),
            pltpu.VMEM((2, CH, DN), jnp.float32),
            pltpu.VMEM_SHARED((NN, DN), jnp.float32),
            pltpu.SemaphoreType.DMA,
        ],
    )
    def scatter_k(u0_hbm, d_hbm, init_hbm, out_hbm,
                  idx0, idx1, upd, acc, sem):
        c = lax.axis_index("c")
        s = lax.axis_index("s")
        wid = s * NC + c

        pltpu.sync_copy(init_hbm.at[c, pl.ds(s * ZR, ZR)],
                        acc.at[pl.ds(s * ZR, ZR)])
        @pl.when(s == NS - 1)
        def _():
            tail = NN - NS * ZR
            pltpu.sync_copy(init_hbm.at[c, pl.ds(NS * ZR, tail)],
                            acc.at[pl.ds(NS * ZR, tail)])
        plsc.subcore_barrier()

        for u_hbm, (row0, nrows) in zip((u0_hbm,), my_slices):
            rows_base = nrows // NW
            rows_extra = nrows % NW
            base = rows_base * wid + jnp.minimum(wid, rows_extra)

            def body(i, _, u_hbm=u_hbm, base=base, row0=row0):
                row = base + 2 * i
                grow = row0 + row
                pltpu.sync_copy(d_hbm.at[pl.ds(grow * CH, CH)], idx0)
                pltpu.sync_copy(d_hbm.at[pl.ds((grow + 1) * CH, CH)], idx1)
                pltpu.sync_copy(u_hbm.at[pl.ds(row, 2)], upd)
                a0 = pltpu.async_copy(upd.at[0], acc.at[idx0], sem, add=True)
                a1 = pltpu.async_copy(upd.at[1], acc.at[idx1], sem, add=True)
                a0.wait(); a1.wait()
                return 0

            lax.fori_loop(0, rows_base // 2, body, 0)

            if rows_extra:
                @pl.when(wid < rows_extra)
                def _(u_hbm=u_hbm, base=base, row0=row0,
                      rows_base=rows_base):
                    row = base + rows_base
                    grow = row0 + row
                    pltpu.sync_copy(d_hbm.at[pl.ds(grow * CH, CH)], idx0)
                    pltpu.sync_copy(u_hbm.at[pl.ds(row, 1)],
                                    upd.at[pl.ds(0, 1)])
                    pltpu.sync_copy(upd.at[0], acc.at[idx0], add=True)

        plsc.subcore_barrier()
        pltpu.sync_copy(acc.at[pl.ds(s * ZR, ZR)], out_hbm.at[c, pl.ds(s * ZR, ZR)])
        @pl.when(s == NS - 1)
        def _():
            tail = NN - NS * ZR
            pltpu.sync_copy(acc.at[pl.ds(NS * ZR, tail)],
                            out_hbm.at[c, pl.ds(NS * ZR, tail)])

    return scatter_k


_sc_gathers = tuple(_make_sc_gather(DN, r0, nr) for r0, nr in SLICES)
_sc_gathers_pack = tuple(_make_sc_gather_pack(r0, nr) for r0, nr in SLICES)
_sc_scatters = tuple(_make_sc_scatter((k,)) for k in range(len(SLICES)))


def _dot0(a, b):
    return lax.dot_general(a, b, (((0,), (0,)), ((), ())),
                           preferred_element_type=jnp.float32)


def _node_msg_body(gs, gd, eft, snwt, w1s, w1e, w1d, b1, w2, b2,
                   g1s, g1e, g1d, c1, g2, c2, ww, bw, out):
    f32 = jnp.float32
    xs = gs[...]
    xd = gd[...]
    et = eft[...]
    pre_h = (jnp.dot(xs, w1s[...], preferred_element_type=f32)
             + _dot0(et, w1e[...])
             + jnp.dot(xd, w1d[...], preferred_element_type=f32) + b1[...])
    pre_g = (jnp.dot(xs, g1s[...], preferred_element_type=f32)
             + _dot0(et, g1e[...])
             + jnp.dot(xd, g1d[...], preferred_element_type=f32) + c1[...])
    h2 = _silu(jnp.dot(_silu(pre_h), w2[...], preferred_element_type=f32) + b2[...])
    gg = jax.nn.sigmoid(jnp.dot(_silu(pre_g), g2[...], preferred_element_type=f32) + c2[...])
    w = jax.nn.sigmoid(_dot0(snwt[...], ww[...]) + bw[...])
    out[...] = h2 * gg * w


def _tc_node_msg(edge0, n_edges, gs, gd, eft, snwt, w1s, w1e, w1d, b1, w2, b2,
                 g1s, g1e, g1d, c1, g2, c2, ww, bw):
    BN = 1280
    grid = n_edges // BN
    blk0 = edge0 // BN
    row = lambda i: (i, 0)
    col = lambda i: (0, i + blk0)
    full = lambda i: (0, 0)

    def wspec(a):
        return pl.BlockSpec(a.shape, full)

    return pl.pallas_call(
        _node_msg_body,
        grid=(grid,),
        in_specs=[
            pl.BlockSpec((BN, DN), row),
            pl.BlockSpec((BN, DN), row),
            pl.BlockSpec((DE, BN), col),
            pl.BlockSpec((DW, BN), col),
            wspec(w1s), wspec(w1e), wspec(w1d), wspec(b1),
            wspec(w2), wspec(b2),
            wspec(g1s), wspec(g1e), wspec(g1d), wspec(c1),
            wspec(g2), wspec(c2), wspec(ww), wspec(bw),
        ],
        out_specs=pl.BlockSpec((BN, DN), row),
        out_shape=jax.ShapeDtypeStruct((n_edges, DN), jnp.float32),
        compiler_params=pltpu.CompilerParams(
            dimension_semantics=("arbitrary",)),
    )(gs, gd, eft, snwt, w1s, w1e, w1d, b1, w2, b2,
      g1s, g1e, g1d, c1, g2, c2, ww, bw)


def _node_out_body(nf, a0, a1, ew1s, eg1s, ew1d, eg1d, nn_out, te_out):
    f32 = jnp.float32
    nn = nf[...] + a0[...] + a1[...]
    nn_out[...] = nn
    te_out[...] = jnp.concatenate(
        [jnp.dot(nn, ew1s[...], preferred_element_type=f32),
         jnp.dot(nn, eg1s[...], preferred_element_type=f32),
         jnp.dot(nn, ew1d[...], preferred_element_type=f32),
         jnp.dot(nn, eg1d[...], preferred_element_type=f32)], axis=1)


def _tc_node_out(nf, a0, a1, ew1s, eg1s, ew1d, eg1d):
    BR = 1000
    grid = NN // BR
    row = lambda i: (i, 0)
    full = lambda i: (0, 0)

    def wspec(a):
        return pl.BlockSpec(a.shape, full)

    return pl.pallas_call(
        _node_out_body,
        grid=(grid,),
        in_specs=[
            pl.BlockSpec((BR, DN), row),
            pl.BlockSpec((BR, DN), row),
            pl.BlockSpec((BR, DN), row),
            wspec(ew1s), wspec(eg1s), wspec(ew1d), wspec(eg1d),
        ],
        out_specs=[
            pl.BlockSpec((BR, DN), row),
            pl.BlockSpec((BR, DN), row),
        ],
        out_shape=[
            jax.ShapeDtypeStruct((NN, DN), jnp.float32),
            jax.ShapeDtypeStruct((NN, DN), jnp.float32),
        ],
        compiler_params=pltpu.CompilerParams(
            dimension_semantics=("arbitrary",)),
    )(nf, a0, a1, ew1s, eg1s, ew1d, eg1d)


def _edge_out_body(ge, eft, sewt, ew1e, eg1e, eb1, eg1, ew2, eb2,
                   eg2, eg2b, eww, ebw, out):
    f32 = jnp.float32
    et = eft[...]
    g = ge[...]
    pre_h = (g[:, :32] + g[:, 64:96] + _dot0(et, ew1e[...]) + eb1[...])
    pre_g = (g[:, 32:64] + g[:, 96:128] + _dot0(et, eg1e[...]) + eg1[...])
    h2 = _silu(jnp.dot(_silu(pre_h), ew2[...], preferred_element_type=f32) + eb2[...])
    g2 = jax.nn.sigmoid(jnp.dot(_silu(pre_g), eg2[...], preferred_element_type=f32) + eg2b[...])
    ew = jax.nn.sigmoid(_dot0(sewt[...], eww[...]) + ebw[...])
    msg = h2 * g2 * ew
    out[...] = et + jnp.swapaxes(msg, 0, 1)


def _tc_edge_out(edge0, n_edges, ge, eft, sewt, ew1e, eg1e, eb1, eg1,
                 ew2, eb2, eg2, eg2b, eww, ebw):
    BN = 2560
    grid = n_edges // BN
    blk0 = edge0 // BN
    row = lambda i: (i, 0)
    col = lambda i: (0, i + blk0)
    ocol = lambda i: (0, i)
    full = lambda i: (0, 0)

    def wspec(a):
        return pl.BlockSpec(a.shape, full)

    return pl.pallas_call(
        _edge_out_body,
        grid=(grid,),
        in_specs=[
            pl.BlockSpec((BN, DN), row),
            pl.BlockSpec((DE, BN), col),
            pl.BlockSpec((DW, BN), col),
            wspec(ew1e), wspec(eg1e), wspec(eb1), wspec(eg1),
            wspec(ew2), wspec(eb2), wspec(eg2), wspec(eg2b),
            wspec(eww), wspec(ebw),
        ],
        out_specs=pl.BlockSpec((DE, BN), ocol),
        out_shape=jax.ShapeDtypeStruct((DE, n_edges), jnp.float32),
        compiler_params=pltpu.CompilerParams(
            dimension_semantics=("arbitrary",)),
    )(ge, eft, sewt, ew1e, eg1e, eb1, eg1, ew2, eb2, eg2, eg2b, eww, ebw)


def kernel(node_features, edge_features, edge_index, shared_node_weights,
           shared_edge_weights, nW1, nb1, nW2, nb2, nG1, ng1, nG2, ng2,
           nWw, nbw, eW1, eb1, eW2, eb2, eG1, eg1, eG2, eg2, eWw, ebw):
    f32 = jnp.float32
    src = edge_index[0]
    dst = edge_index[1]
    eft = jnp.swapaxes(edge_features, 0, 1)
    snwt = jnp.swapaxes(shared_node_weights, 0, 1)
    sewt = jnp.swapaxes(shared_edge_weights, 0, 1)

    contribs = []
    for k, (r0, nr) in enumerate(SLICES):
        gs3, gd3 = _sc_gathers[k](node_features, src, dst)
        contrib_k = _tc_node_msg(
            r0 * CH, nr * CH,
            jnp.reshape(gs3, (nr * CH, DN)), jnp.reshape(gd3, (nr * CH, DN)),
            eft, snwt,
            nW1[:DN], nW1[DN:DN + DE], nW1[DN + DE:], nb1.reshape(1, DN),
            nW2, nb2.reshape(1, DN),
            nG1[:DN], nG1[DN:DN + DE], nG1[DN + DE:], ng1.reshape(1, DN),
            nG2, ng2.reshape(1, DN), nWw, nbw.reshape(1, DN))
        contribs.append(jnp.reshape(contrib_k, (nr, CH, DN)))

    aggp = jnp.zeros((NC, NN, DN), f32)
    for k in range(len(SLICES)):
        aggp = _sc_scatters[k](contribs[k], dst, aggp)

    nn, te = _tc_node_out(node_features, aggp[0], aggp[1],
                          eW1[:DN], eG1[:DN], eW1[DN + DE:], eG1[DN + DE:])

    ne_parts = []
    for k, (r0, nr) in enumerate(SLICES):
        ge3 = _sc_gathers_pack[k](te, src, dst)
        ne_parts.append(_tc_edge_out(
            r0 * CH, nr * CH,
            jnp.reshape(ge3, (nr * CH, DN)),
            eft, sewt,
            eW1[DN:DN + DE], eG1[DN:DN + DE], eb1.reshape(1, 32),
            eg1.reshape(1, 32), eW2, eb2.reshape(1, DE), eG2,
            eg2.reshape(1, DE), eWw, ebw.reshape(1, DE)))

    new_edges_t = jnp.concatenate(ne_parts, axis=1)
    return nn, jnp.swapaxes(new_edges_t, 0, 1)


# speedup vs baseline: 4.1029x; 1.0239x over previous
import functools

import jax
import jax.numpy as jnp
from jax import lax
from jax.experimental import pallas as pl
from jax.experimental.pallas import tpu as pltpu
from jax.experimental.pallas import tpu_sc as plsc

NN = 10000
NE = 320000
DN = 128
DE = 16
DW = 32

NC = 2
NS = 16
NW = NC * NS
CH = 128
NROW = NE // CH
ROWS_BASE = NROW // NW
ROWS_EXTRA = NROW % NW


def _silu(x):
    return x * jax.nn.sigmoid(x)


SLICES = ((0, 320), (320, 320), (640, 640), (1280, 640), (1920, 580))
SCATTER_GROUPS = ((0, 1), (2,), (3,), (4,))


def _make_sc_gather(width, row0, nrows):
    mesh = plsc.VectorSubcoreMesh(core_axis_name="c", subcore_axis_name="s")
    rows_base = nrows // NW
    rows_extra = nrows % NW
    assert rows_base % 2 == 0

    @functools.partial(
        pl.kernel,
        mesh=mesh,
        out_type=(
            jax.ShapeDtypeStruct((nrows, CH, width), jnp.float32),
            jax.ShapeDtypeStruct((nrows, CH, width), jnp.float32),
        ),
        scratch_types=[
            pltpu.VMEM((2 * CH,), jnp.int32),
            pltpu.VMEM((2 * CH,), jnp.int32),
            pltpu.VMEM((2, CH, width), jnp.float32),
            pltpu.VMEM((2, CH, width), jnp.float32),
            pltpu.SemaphoreType.DMA,
            pltpu.SemaphoreType.DMA,
        ],
    )
    def gather_k(tab_hbm, s_hbm, d_hbm, gs_hbm, gd_hbm,
                 idxs, idxd, rs, rd, gsem, wsem):
        c = lax.axis_index("c")
        s = lax.axis_index("s")
        wid = s * NC + c
        base = rows_base * wid + jnp.minimum(wid, rows_extra)

        def do_pair(row, first):
            grow = row0 + row
            pltpu.sync_copy(s_hbm.at[pl.ds(grow * CH, 2 * CH)], idxs)
            pltpu.sync_copy(d_hbm.at[pl.ds(grow * CH, 2 * CH)], idxd)
            @pl.when(jnp.logical_not(first))
            def _():
                pltpu.make_async_copy(rs, gs_hbm.at[pl.ds(row, 2)], wsem).wait()
                pltpu.make_async_copy(rd, gd_hbm.at[pl.ds(row, 2)], wsem).wait()
            d0 = pltpu.async_copy(tab_hbm.at[idxs.at[pl.ds(0, CH)]], rs.at[0], gsem)
            d1 = pltpu.async_copy(tab_hbm.at[idxs.at[pl.ds(CH, CH)]], rs.at[1], gsem)
            d2 = pltpu.async_copy(tab_hbm.at[idxd.at[pl.ds(0, CH)]], rd.at[0], gsem)
            d3 = pltpu.async_copy(tab_hbm.at[idxd.at[pl.ds(CH, CH)]], rd.at[1], gsem)
            d0.wait(); d1.wait(); d2.wait(); d3.wait()
            pltpu.async_copy(rs, gs_hbm.at[pl.ds(row, 2)], wsem)
            pltpu.async_copy(rd, gd_hbm.at[pl.ds(row, 2)], wsem)

        def body(i, _):
            do_pair(base + 2 * i, i == 0)
            return 0

        lax.fori_loop(0, rows_base // 2, body, 0)
        pltpu.make_async_copy(rs, gs_hbm.at[pl.ds(base, 2)], wsem).wait()
        pltpu.make_async_copy(rd, gd_hbm.at[pl.ds(base, 2)], wsem).wait()

        if rows_extra:
            @pl.when(wid < rows_extra)
            def _():
                row = base + rows_base
                grow = row0 + row
                pltpu.sync_copy(s_hbm.at[pl.ds(grow * CH, CH)],
                                idxs.at[pl.ds(0, CH)])
                pltpu.sync_copy(d_hbm.at[pl.ds(grow * CH, CH)],
                                idxd.at[pl.ds(0, CH)])
                d0 = pltpu.async_copy(tab_hbm.at[idxs.at[pl.ds(0, CH)]],
                                      rs.at[0], gsem)
                d1 = pltpu.async_copy(tab_hbm.at[idxd.at[pl.ds(0, CH)]],
                                      rd.at[0], gsem)
                d0.wait(); d1.wait()
                pltpu.sync_copy(rs.at[pl.ds(0, 1)], gs_hbm.at[pl.ds(row, 1)])
                pltpu.sync_copy(rd.at[pl.ds(0, 1)], gd_hbm.at[pl.ds(row, 1)])

    return gather_k


def _make_sc_gather_pack(row0, nrows):
    W = DN
    H = W // 2
    mesh = plsc.VectorSubcoreMesh(core_axis_name="c", subcore_axis_name="s")
    ch_base = nrows // NW
    ch_extra = nrows % NW
    assert ch_base % 2 == 0

    @functools.partial(
        pl.kernel,
        mesh=mesh,
        out_type=jax.ShapeDtypeStruct((nrows, CH, W), jnp.float32),
        scratch_types=[
            pltpu.VMEM((CH,), jnp.int32),
            pltpu.VMEM((CH,), jnp.int32),
            pltpu.VMEM((CH,), jnp.int32),
            pltpu.VMEM((CH,), jnp.int32),
            pltpu.VMEM((CH, W), jnp.float32),
            pltpu.VMEM((CH, W), jnp.float32),
            pltpu.VMEM((CH, W), jnp.float32),
            pltpu.VMEM((CH, W), jnp.float32),
            pltpu.VMEM((1, CH, W), jnp.float32),
            pltpu.VMEM((1, CH, W), jnp.float32),
            pltpu.SemaphoreType.DMA,
            pltpu.SemaphoreType.DMA,
        ],
    )
    def gather_k(tab_hbm, s_hbm, d_hbm, g_hbm,
                 is0, is1, id0, id1, rs0, rs1, rd0, rd1, pk0, pk1,
                 gsem, wsem):
        c = lax.axis_index("c")
        s = lax.axis_index("s")
        wid = s * NC + c
        base = ch_base * wid + jnp.minimum(wid, ch_extra)
        isb = (is0, is1)
        idb = (id0, id1)
        rsb = (rs0, rs1)
        rdb = (rd0, rd1)
        pkb = (pk0, pk1)

        def fire(row, p):
            grow = row0 + row
            pltpu.sync_copy(s_hbm.at[pl.ds(grow * CH, CH)], isb[p])
            pltpu.sync_copy(d_hbm.at[pl.ds(grow * CH, CH)], idb[p])
            pltpu.async_copy(tab_hbm.at[isb[p]], rsb[p], gsem)
            pltpu.async_copy(tab_hbm.at[idb[p]], rdb[p], gsem)

        def merge_wb(row, p, first):
            pltpu.make_async_copy(tab_hbm.at[isb[p]], rsb[p], gsem).wait()
            pltpu.make_async_copy(tab_hbm.at[idb[p]], rdb[p], gsem).wait()
            @pl.when(jnp.logical_not(first))
            def _():
                pltpu.make_async_copy(pkb[p], g_hbm.at[pl.ds(row, 1)],
                                      wsem).wait()
            rs, rd, pk = rsb[p], rdb[p], pkb[p]

            def mrow(r, _):
                for k in range(H // 16):
                    pk[0, r, pl.ds(k * 16, 16)] = rs[r, pl.ds(k * 16, 16)]
                    pk[0, r, pl.ds(H + k * 16, 16)] = rd[r, pl.ds(H + k * 16, 16)]
                return 0

            lax.fori_loop(0, CH, mrow, 0)
            pltpu.async_copy(pk, g_hbm.at[pl.ds(row, 1)], wsem)

        fire(base, 0)

        def body(i, _):
            r0 = base + 2 * i
            fire(r0 + 1, 1)
            merge_wb(r0, 0, i == 0)
            @pl.when(2 * i + 2 < ch_base)
            def _():
                fire(r0 + 2, 0)
            merge_wb(r0 + 1, 1, i == 0)
            return 0

        lax.fori_loop(0, ch_base // 2, body, 0)
        pltpu.make_async_copy(pk0, g_hbm.at[pl.ds(base, 1)], wsem).wait()
        pltpu.make_async_copy(pk1, g_hbm.at[pl.ds(base, 1)], wsem).wait()

        if ch_extra:
            @pl.when(wid < ch_extra)
            def _():
                row = base + ch_base
                fire(row, 0)
                pltpu.make_async_copy(tab_hbm.at[is0], rs0, gsem).wait()
                pltpu.make_async_copy(tab_hbm.at[id0], rd0, gsem).wait()

                def mrow(r, _):
                    for k in range(H // 16):
                        pk0[0, r, pl.ds(k * 16, 16)] = rs0[r, pl.ds(k * 16, 16)]
                        pk0[0, r, pl.ds(H + k * 16, 16)] = rd0[r, pl.ds(H + k * 16, 16)]
                    return 0

                lax.fori_loop(0, CH, mrow, 0)
                pltpu.sync_copy(pk0, g_hbm.at[pl.ds(row, 1)])

    return gather_k


def _make_sc_scatter(slice_ids):
    mesh = plsc.VectorSubcoreMesh(core_axis_name="c", subcore_axis_name="s")
    ZR = 624
    my_slices = [SLICES[i] for i in slice_ids]

    @functools.partial(
        pl.kernel,
        mesh=mesh,
        out_type=jax.ShapeDtypeStruct((NC, NN, DN), jnp.float32),
        scratch_types=[
            pltpu.VMEM((CH,), jnp.int32),
            pltpu.VMEM((CH,), jnp.int32),
            pltpu.VMEM((2, CH, DN), jnp.float32),
            pltpu.VMEM_SHARED((NN, DN), jnp.float32),
            pltpu.SemaphoreType.DMA,
        ],
    )
    def scatter_k(*args):
        u_hbms = args[:len(my_slices)]
        d_hbm, init_hbm, out_hbm, idx0, idx1, upd, acc, sem = \
            args[len(my_slices):]
        c = lax.axis_index("c")
        s = lax.axis_index("s")
        wid = s * NC + c

        pltpu.sync_copy(init_hbm.at[c, pl.ds(s * ZR, ZR)],
                        acc.at[pl.ds(s * ZR, ZR)])
        @pl.when(s == NS - 1)
        def _():
            tail = NN - NS * ZR
            pltpu.sync_copy(init_hbm.at[c, pl.ds(NS * ZR, tail)],
                            acc.at[pl.ds(NS * ZR, tail)])
        plsc.subcore_barrier()

        for u_hbm, (row0, nrows) in zip(u_hbms, my_slices):
            rows_base = nrows // NW
            rows_extra = nrows % NW
            base = rows_base * wid + jnp.minimum(wid, rows_extra)

            def body(i, _, u_hbm=u_hbm, base=base, row0=row0):
                row = base + 2 * i
                grow = row0 + row
                pltpu.sync_copy(d_hbm.at[pl.ds(grow * CH, CH)], idx0)
                pltpu.sync_copy(d_hbm.at[pl.ds((grow + 1) * CH, CH)], idx1)
                pltpu.sync_copy(u_hbm.at[pl.ds(row, 2)], upd)
                a0 = pltpu.async_copy(upd.at[0], acc.at[idx0], sem, add=True)
                a1 = pltpu.async_copy(upd.at[1], acc.at[idx1], sem, add=True)
                a0.wait(); a1.wait()
                return 0

            lax.fori_loop(0, rows_base // 2, body, 0)

            if rows_extra:
                @pl.when(wid < rows_extra)
                def _(u_hbm=u_hbm, base=base, row0=row0,
                      rows_base=rows_base):
                    row = base + rows_base
                    grow = row0 + row
                    pltpu.sync_copy(d_hbm.at[pl.ds(grow * CH, CH)], idx0)
                    pltpu.sync_copy(u_hbm.at[pl.ds(row, 1)],
                                    upd.at[pl.ds(0, 1)])
                    pltpu.sync_copy(upd.at[0], acc.at[idx0], add=True)

        plsc.subcore_barrier()
        pltpu.sync_copy(acc.at[pl.ds(s * ZR, ZR)], out_hbm.at[c, pl.ds(s * ZR, ZR)])
        @pl.when(s == NS - 1)
        def _():
            tail = NN - NS * ZR
            pltpu.sync_copy(acc.at[pl.ds(NS * ZR, tail)],
                            out_hbm.at[c, pl.ds(NS * ZR, tail)])

    return scatter_k


_sc_gathers = tuple(_make_sc_gather(DN, r0, nr) for r0, nr in SLICES)
_sc_gathers_pack = tuple(_make_sc_gather_pack(r0, nr) for r0, nr in SLICES)
_sc_scatters = tuple(_make_sc_scatter(g) for g in SCATTER_GROUPS)


def _dot0(a, b):
    return lax.dot_general(a, b, (((0,), (0,)), ((), ())),
                           preferred_element_type=jnp.float32)


def _node_msg_body(gs, gd, eft, snwt, w1s, w1e, w1d, b1, w2, b2,
                   g1s, g1e, g1d, c1, g2, c2, ww, bw, out):
    f32 = jnp.float32
    xs = gs[...]
    xd = gd[...]
    et = eft[...]
    pre_h = (jnp.dot(xs, w1s[...], preferred_element_type=f32)
             + _dot0(et, w1e[...])
             + jnp.dot(xd, w1d[...], preferred_element_type=f32) + b1[...])
    pre_g = (jnp.dot(xs, g1s[...], preferred_element_type=f32)
             + _dot0(et, g1e[...])
             + jnp.dot(xd, g1d[...], preferred_element_type=f32) + c1[...])
    h2 = _silu(jnp.dot(_silu(pre_h), w2[...], preferred_element_type=f32) + b2[...])
    gg = jax.nn.sigmoid(jnp.dot(_silu(pre_g), g2[...], preferred_element_type=f32) + c2[...])
    w = jax.nn.sigmoid(_dot0(snwt[...], ww[...]) + bw[...])
    out[...] = h2 * gg * w


def _tc_node_msg(edge0, n_edges, gs, gd, eft, snwt, w1s, w1e, w1d, b1, w2, b2,
                 g1s, g1e, g1d, c1, g2, c2, ww, bw):
    BN = 1280
    grid = n_edges // BN
    blk0 = edge0 // BN
    row = lambda i: (i, 0)
    col = lambda i: (0, i + blk0)
    full = lambda i: (0, 0)

    def wspec(a):
        return pl.BlockSpec(a.shape, full)

    return pl.pallas_call(
        _node_msg_body,
        grid=(grid,),
        in_specs=[
            pl.BlockSpec((BN, DN), row),
            pl.BlockSpec((BN, DN), row),
            pl.BlockSpec((DE, BN), col),
            pl.BlockSpec((DW, BN), col),
            wspec(w1s), wspec(w1e), wspec(w1d), wspec(b1),
            wspec(w2), wspec(b2),
            wspec(g1s), wspec(g1e), wspec(g1d), wspec(c1),
            wspec(g2), wspec(c2), wspec(ww), wspec(bw),
        ],
        out_specs=pl.BlockSpec((BN, DN), row),
        out_shape=jax.ShapeDtypeStruct((n_edges, DN), jnp.float32),
        compiler_params=pltpu.CompilerParams(
            dimension_semantics=("arbitrary",)),
    )(gs, gd, eft, snwt, w1s, w1e, w1d, b1, w2, b2,
      g1s, g1e, g1d, c1, g2, c2, ww, bw)


def _node_out_body(nf, a0, a1, ew1s, eg1s, ew1d, eg1d, nn_out, te_out):
    f32 = jnp.float32
    nn = nf[...] + a0[...] + a1[...]
    nn_out[...] = nn
    te_out[...] = jnp.concatenate(
        [jnp.dot(nn, ew1s[...], preferred_element_type=f32),
         jnp.dot(nn, eg1s[...], preferred_element_type=f32),
         jnp.dot(nn, ew1d[...], preferred_element_type=f32),
         jnp.dot(nn, eg1d[...], preferred_element_type=f32)], axis=1)


def _tc_node_out(nf, a0, a1, ew1s, eg1s, ew1d, eg1d):
    BR = 1000
    grid = NN // BR
    row = lambda i: (i, 0)
    full = lambda i: (0, 0)

    def wspec(a):
        return pl.BlockSpec(a.shape, full)

    return pl.pallas_call(
        _node_out_body,
        grid=(grid,),
        in_specs=[
            pl.BlockSpec((BR, DN), row),
            pl.BlockSpec((BR, DN), row),
            pl.BlockSpec((BR, DN), row),
            wspec(ew1s), wspec(eg1s), wspec(ew1d), wspec(eg1d),
        ],
        out_specs=[
            pl.BlockSpec((BR, DN), row),
            pl.BlockSpec((BR, DN), row),
        ],
        out_shape=[
            jax.ShapeDtypeStruct((NN, DN), jnp.float32),
            jax.ShapeDtypeStruct((NN, DN), jnp.float32),
        ],
        compiler_params=pltpu.CompilerParams(
            dimension_semantics=("arbitrary",)),
    )(nf, a0, a1, ew1s, eg1s, ew1d, eg1d)


def _edge_out_body(ge, eft, sewt, ew1e, eg1e, eb1, eg1, ew2, eb2,
                   eg2, eg2b, eww, ebw, out):
    f32 = jnp.float32
    et = eft[...]
    g = ge[...]
    pre_h = (g[:, :32] + g[:, 64:96] + _dot0(et, ew1e[...]) + eb1[...])
    pre_g = (g[:, 32:64] + g[:, 96:128] + _dot0(et, eg1e[...]) + eg1[...])
    h2 = _silu(jnp.dot(_silu(pre_h), ew2[...], preferred_element_type=f32) + eb2[...])
    g2 = jax.nn.sigmoid(jnp.dot(_silu(pre_g), eg2[...], preferred_element_type=f32) + eg2b[...])
    ew = jax.nn.sigmoid(_dot0(sewt[...], eww[...]) + ebw[...])
    msg = h2 * g2 * ew
    out[...] = et + jnp.swapaxes(msg, 0, 1)


def _tc_edge_out(edge0, n_edges, ge, eft, sewt, ew1e, eg1e, eb1, eg1,
                 ew2, eb2, eg2, eg2b, eww, ebw):
    BN = 2560
    grid = n_edges // BN
    blk0 = edge0 // BN
    row = lambda i: (i, 0)
    col = lambda i: (0, i + blk0)
    ocol = lambda i: (0, i)
    full = lambda i: (0, 0)

    def wspec(a):
        return pl.BlockSpec(a.shape, full)

    return pl.pallas_call(
        _edge_out_body,
        grid=(grid,),
        in_specs=[
            pl.BlockSpec((BN, DN), row),
            pl.BlockSpec((DE, BN), col),
            pl.BlockSpec((DW, BN), col),
            wspec(ew1e), wspec(eg1e), wspec(eb1), wspec(eg1),
            wspec(ew2), wspec(eb2), wspec(eg2), wspec(eg2b),
            wspec(eww), wspec(ebw),
        ],
        out_specs=pl.BlockSpec((DE, BN), ocol),
        out_shape=jax.ShapeDtypeStruct((DE, n_edges), jnp.float32),
        compiler_params=pltpu.CompilerParams(
            dimension_semantics=("arbitrary",)),
    )(ge, eft, sewt, ew1e, eg1e, eb1, eg1, ew2, eb2, eg2, eg2b, eww, ebw)


def kernel(node_features, edge_features, edge_index, shared_node_weights,
           shared_edge_weights, nW1, nb1, nW2, nb2, nG1, ng1, nG2, ng2,
           nWw, nbw, eW1, eb1, eW2, eb2, eG1, eg1, eG2, eg2, eWw, ebw):
    f32 = jnp.float32
    src = edge_index[0]
    dst = edge_index[1]
    eft = jnp.swapaxes(edge_features, 0, 1)
    snwt = jnp.swapaxes(shared_node_weights, 0, 1)
    sewt = jnp.swapaxes(shared_edge_weights, 0, 1)

    contribs = []
    for k, (r0, nr) in enumerate(SLICES):
        gs3, gd3 = _sc_gathers[k](node_features, src, dst)
        contrib_k = _tc_node_msg(
            r0 * CH, nr * CH,
            jnp.reshape(gs3, (nr * CH, DN)), jnp.reshape(gd3, (nr * CH, DN)),
            eft, snwt,
            nW1[:DN], nW1[DN:DN + DE], nW1[DN + DE:], nb1.reshape(1, DN),
            nW2, nb2.reshape(1, DN),
            nG1[:DN], nG1[DN:DN + DE], nG1[DN + DE:], ng1.reshape(1, DN),
            nG2, ng2.reshape(1, DN), nWw, nbw.reshape(1, DN))
        contribs.append(jnp.reshape(contrib_k, (nr, CH, DN)))

    aggp = jnp.zeros((NC, NN, DN), f32)
    for g, grp in enumerate(SCATTER_GROUPS):
        aggp = _sc_scatters[g](*[contribs[k] for k in grp], dst, aggp)

    nn, te = _tc_node_out(node_features, aggp[0], aggp[1],
                          eW1[:DN], eG1[:DN], eW1[DN + DE:], eG1[DN + DE:])

    ne_parts = []
    for k, (r0, nr) in enumerate(SLICES):
        ge3 = _sc_gathers_pack[k](te, src, dst)
        ne_parts.append(_tc_edge_out(
            r0 * CH, nr * CH,
            jnp.reshape(ge3, (nr * CH, DN)),
            eft, sewt,
            eW1[DN:DN + DE], eG1[DN:DN + DE], eb1.reshape(1, 32),
            eg1.reshape(1, 32), eW2, eb2.reshape(1, DE), eG2,
            eg2.reshape(1, DE), eWw, ebw.reshape(1, DE)))

    new_edges_t = jnp.concatenate(ne_parts, axis=1)
    return nn, jnp.swapaxes(new_edges_t, 0, 1)


# speedup vs baseline: 4.1083x; 1.0013x over previous
import functools

import jax
import jax.numpy as jnp
from jax import lax
from jax.experimental import pallas as pl
from jax.experimental.pallas import tpu as pltpu
from jax.experimental.pallas import tpu_sc as plsc

NN = 10000
NE = 320000
DN = 128
DE = 16
DW = 32

NC = 2
NS = 16
NW = NC * NS
CH = 128
NROW = NE // CH
ROWS_BASE = NROW // NW
ROWS_EXTRA = NROW % NW


def _silu(x):
    return x * jax.nn.sigmoid(x)


SLICES = ((0, 320), (320, 320), (640, 640), (1280, 640), (1920, 580))
SCATTER_GROUPS = ((0, 1), (2,), (3,), (4,))


def _make_sc_gather(width, row0, nrows):
    mesh = plsc.VectorSubcoreMesh(core_axis_name="c", subcore_axis_name="s")
    rows_base = nrows // NW
    rows_extra = nrows % NW
    assert rows_base % 2 == 0

    @functools.partial(
        pl.kernel,
        mesh=mesh,
        out_type=(
            jax.ShapeDtypeStruct((nrows, CH, width), jnp.float32),
            jax.ShapeDtypeStruct((nrows, CH, width), jnp.float32),
        ),
        scratch_types=[
            pltpu.VMEM((2 * CH,), jnp.int32),
            pltpu.VMEM((2 * CH,), jnp.int32),
            pltpu.VMEM((2, CH, width), jnp.float32),
            pltpu.VMEM((2, CH, width), jnp.float32),
            pltpu.SemaphoreType.DMA,
            pltpu.SemaphoreType.DMA,
        ],
    )
    def gather_k(tab_hbm, s_hbm, d_hbm, gs_hbm, gd_hbm,
                 idxs, idxd, rs, rd, gsem, wsem):
        c = lax.axis_index("c")
        s = lax.axis_index("s")
        wid = s * NC + c
        base = rows_base * wid + jnp.minimum(wid, rows_extra)

        def do_pair(row, first):
            grow = row0 + row
            pltpu.sync_copy(s_hbm.at[pl.ds(grow * CH, 2 * CH)], idxs)
            pltpu.sync_copy(d_hbm.at[pl.ds(grow * CH, 2 * CH)], idxd)
            @pl.when(jnp.logical_not(first))
            def _():
                pltpu.make_async_copy(rs, gs_hbm.at[pl.ds(row, 2)], wsem).wait()
                pltpu.make_async_copy(rd, gd_hbm.at[pl.ds(row, 2)], wsem).wait()
            d0 = pltpu.async_copy(tab_hbm.at[idxs.at[pl.ds(0, CH)]], rs.at[0], gsem)
            d1 = pltpu.async_copy(tab_hbm.at[idxs.at[pl.ds(CH, CH)]], rs.at[1], gsem)
            d2 = pltpu.async_copy(tab_hbm.at[idxd.at[pl.ds(0, CH)]], rd.at[0], gsem)
            d3 = pltpu.async_copy(tab_hbm.at[idxd.at[pl.ds(CH, CH)]], rd.at[1], gsem)
            d0.wait(); d1.wait(); d2.wait(); d3.wait()
            pltpu.async_copy(rs, gs_hbm.at[pl.ds(row, 2)], wsem)
            pltpu.async_copy(rd, gd_hbm.at[pl.ds(row, 2)], wsem)

        def body(i, _):
            do_pair(base + 2 * i, i == 0)
            return 0

        lax.fori_loop(0, rows_base // 2, body, 0)
        pltpu.make_async_copy(rs, gs_hbm.at[pl.ds(base, 2)], wsem).wait()
        pltpu.make_async_copy(rd, gd_hbm.at[pl.ds(base, 2)], wsem).wait()

        if rows_extra:
            @pl.when(wid < rows_extra)
            def _():
                row = base + rows_base
                grow = row0 + row
                pltpu.sync_copy(s_hbm.at[pl.ds(grow * CH, CH)],
                                idxs.at[pl.ds(0, CH)])
                pltpu.sync_copy(d_hbm.at[pl.ds(grow * CH, CH)],
                                idxd.at[pl.ds(0, CH)])
                d0 = pltpu.async_copy(tab_hbm.at[idxs.at[pl.ds(0, CH)]],
                                      rs.at[0], gsem)
                d1 = pltpu.async_copy(tab_hbm.at[idxd.at[pl.ds(0, CH)]],
                                      rd.at[0], gsem)
                d0.wait(); d1.wait()
                pltpu.sync_copy(rs.at[pl.ds(0, 1)], gs_hbm.at[pl.ds(row, 1)])
                pltpu.sync_copy(rd.at[pl.ds(0, 1)], gd_hbm.at[pl.ds(row, 1)])

    return gather_k


def _make_sc_gather_pack(row0, nrows):
    W = DN
    H = W // 2
    mesh = plsc.VectorSubcoreMesh(core_axis_name="c", subcore_axis_name="s")
    ch_base = nrows // NW
    ch_extra = nrows % NW
    assert ch_base % 2 == 0

    @functools.partial(
        pl.kernel,
        mesh=mesh,
        out_type=jax.ShapeDtypeStruct((nrows, CH, W), jnp.float32),
        scratch_types=[
            pltpu.VMEM((CH,), jnp.int32),
            pltpu.VMEM((CH,), jnp.int32),
            pltpu.VMEM((CH,), jnp.int32),
            pltpu.VMEM((CH,), jnp.int32),
            pltpu.VMEM((CH, W), jnp.float32),
            pltpu.VMEM((CH, W), jnp.float32),
            pltpu.VMEM((CH, W), jnp.float32),
            pltpu.VMEM((CH, W), jnp.float32),
            pltpu.VMEM((1, CH, W), jnp.float32),
            pltpu.VMEM((1, CH, W), jnp.float32),
            pltpu.SemaphoreType.DMA,
            pltpu.SemaphoreType.DMA,
        ],
    )
    def gather_k(tab_hbm, s_hbm, d_hbm, g_hbm,
                 is0, is1, id0, id1, rs0, rs1, rd0, rd1, pk0, pk1,
                 gsem, wsem):
        c = lax.axis_index("c")
        s = lax.axis_index("s")
        wid = s * NC + c
        base = ch_base * wid + jnp.minimum(wid, ch_extra)
        isb = (is0, is1)
        idb = (id0, id1)
        rsb = (rs0, rs1)
        rdb = (rd0, rd1)
        pkb = (pk0, pk1)

        def fire(row, p):
            grow = row0 + row
            pltpu.sync_copy(s_hbm.at[pl.ds(grow * CH, CH)], isb[p])
            pltpu.sync_copy(d_hbm.at[pl.ds(grow * CH, CH)], idb[p])
            pltpu.async_copy(tab_hbm.at[isb[p]], rsb[p], gsem)
            pltpu.async_copy(tab_hbm.at[idb[p]], rdb[p], gsem)

        def merge_wb(row, p, first):
            pltpu.make_async_copy(tab_hbm.at[isb[p]], rsb[p], gsem).wait()
            pltpu.make_async_copy(tab_hbm.at[idb[p]], rdb[p], gsem).wait()
            @pl.when(jnp.logical_not(first))
            def _():
                pltpu.make_async_copy(pkb[p], g_hbm.at[pl.ds(row, 1)],
                                      wsem).wait()
            rs, rd, pk = rsb[p], rdb[p], pkb[p]

            def mrow(r, _):
                for k in range(H // 16):
                    pk[0, r, pl.ds(k * 16, 16)] = rs[r, pl.ds(k * 16, 16)]
                    pk[0, r, pl.ds(H + k * 16, 16)] = rd[r, pl.ds(H + k * 16, 16)]
                return 0

            lax.fori_loop(0, CH, mrow, 0)
            pltpu.async_copy(pk, g_hbm.at[pl.ds(row, 1)], wsem)

        fire(base, 0)

        def body(i, _):
            r0 = base + 2 * i
            fire(r0 + 1, 1)
            merge_wb(r0, 0, i == 0)
            @pl.when(2 * i + 2 < ch_base)
            def _():
                fire(r0 + 2, 0)
            merge_wb(r0 + 1, 1, i == 0)
            return 0

        lax.fori_loop(0, ch_base // 2, body, 0)
        pltpu.make_async_copy(pk0, g_hbm.at[pl.ds(base, 1)], wsem).wait()
        pltpu.make_async_copy(pk1, g_hbm.at[pl.ds(base, 1)], wsem).wait()

        if ch_extra:
            @pl.when(wid < ch_extra)
            def _():
                row = base + ch_base
                fire(row, 0)
                pltpu.make_async_copy(tab_hbm.at[is0], rs0, gsem).wait()
                pltpu.make_async_copy(tab_hbm.at[id0], rd0, gsem).wait()

                def mrow(r, _):
                    for k in range(H // 16):
                        pk0[0, r, pl.ds(k * 16, 16)] = rs0[r, pl.ds(k * 16, 16)]
                        pk0[0, r, pl.ds(H + k * 16, 16)] = rd0[r, pl.ds(H + k * 16, 16)]
                    return 0

                lax.fori_loop(0, CH, mrow, 0)
                pltpu.sync_copy(pk0, g_hbm.at[pl.ds(row, 1)])

    return gather_k


def _make_sc_scatter(slice_ids):
    mesh = plsc.VectorSubcoreMesh(core_axis_name="c", subcore_axis_name="s")
    ZR = 624
    my_slices = [SLICES[i] for i in slice_ids]

    @functools.partial(
        pl.kernel,
        mesh=mesh,
        out_type=jax.ShapeDtypeStruct((NC, NN, DN), jnp.float32),
        scratch_types=[
            pltpu.VMEM((CH,), jnp.int32),
            pltpu.VMEM((CH,), jnp.int32),
            pltpu.VMEM((2, CH, DN), jnp.float32),
            pltpu.VMEM_SHARED((NN, DN), jnp.float32),
            pltpu.SemaphoreType.DMA,
        ],
    )
    def scatter_k(*args):
        u_hbms = args[:len(my_slices)]
        d_hbm, init_hbm, out_hbm, idx0, idx1, upd, acc, sem = \
            args[len(my_slices):]
        c = lax.axis_index("c")
        s = lax.axis_index("s")
        wid = s * NC + c

        pltpu.sync_copy(init_hbm.at[c, pl.ds(s * ZR, ZR)],
                        acc.at[pl.ds(s * ZR, ZR)])
        @pl.when(s == NS - 1)
        def _():
            tail = NN - NS * ZR
            pltpu.sync_copy(init_hbm.at[c, pl.ds(NS * ZR, tail)],
                            acc.at[pl.ds(NS * ZR, tail)])
        plsc.subcore_barrier()

        for u_hbm, (row0, nrows) in zip(u_hbms, my_slices):
            rows_base = nrows // NW
            rows_extra = nrows % NW
            base = rows_base * wid + jnp.minimum(wid, rows_extra)

            def body(i, _, u_hbm=u_hbm, base=base, row0=row0):
                row = base + 2 * i
                grow = row0 + row
                pltpu.sync_copy(d_hbm.at[pl.ds(grow * CH, CH)], idx0)
                pltpu.sync_copy(d_hbm.at[pl.ds((grow + 1) * CH, CH)], idx1)
                pltpu.sync_copy(u_hbm.at[pl.ds(row, 2)], upd)
                a0 = pltpu.async_copy(upd.at[0], acc.at[idx0], sem, add=True)
                a1 = pltpu.async_copy(upd.at[1], acc.at[idx1], sem, add=True)
                a0.wait(); a1.wait()
                return 0

            lax.fori_loop(0, rows_base // 2, body, 0)

            if rows_extra:
                @pl.when(wid < rows_extra)
                def _(u_hbm=u_hbm, base=base, row0=row0,
                      rows_base=rows_base):
                    row = base + rows_base
                    grow = row0 + row
                    pltpu.sync_copy(d_hbm.at[pl.ds(grow * CH, CH)], idx0)
                    pltpu.sync_copy(u_hbm.at[pl.ds(row, 1)],
                                    upd.at[pl.ds(0, 1)])
                    pltpu.sync_copy(upd.at[0], acc.at[idx0], add=True)

        plsc.subcore_barrier()
        pltpu.sync_copy(acc.at[pl.ds(s * ZR, ZR)], out_hbm.at[c, pl.ds(s * ZR, ZR)])
        @pl.when(s == NS - 1)
        def _():
            tail = NN - NS * ZR
            pltpu.sync_copy(acc.at[pl.ds(NS * ZR, tail)],
                            out_hbm.at[c, pl.ds(NS * ZR, tail)])

    return scatter_k


_sc_gathers = tuple(_make_sc_gather(DN, r0, nr) for r0, nr in SLICES)
_sc_gathers_pack = tuple(_make_sc_gather_pack(r0, nr) for r0, nr in SLICES)
_sc_scatters = tuple(_make_sc_scatter(g) for g in SCATTER_GROUPS)


def _dot0(a, b):
    return lax.dot_general(a, b, (((0,), (0,)), ((), ())),
                           preferred_element_type=jnp.float32)


def _node_msg_body(gs, gd, eft, snwt, w1s, w1e, w1d, b1, w2, b2,
                   g1s, g1e, g1d, c1, g2, c2, ww, bw, out):
    f32 = jnp.float32
    xs = gs[...].astype(f32)
    xd = gd[...].astype(f32)
    et = eft[...]
    pre_h = (jnp.dot(xs, w1s[...], preferred_element_type=f32)
             + _dot0(et, w1e[...])
             + jnp.dot(xd, w1d[...], preferred_element_type=f32) + b1[...])
    pre_g = (jnp.dot(xs, g1s[...], preferred_element_type=f32)
             + _dot0(et, g1e[...])
             + jnp.dot(xd, g1d[...], preferred_element_type=f32) + c1[...])
    h2 = _silu(jnp.dot(_silu(pre_h), w2[...], preferred_element_type=f32) + b2[...])
    gg = jax.nn.sigmoid(jnp.dot(_silu(pre_g), g2[...], preferred_element_type=f32) + c2[...])
    w = jax.nn.sigmoid(_dot0(snwt[...], ww[...]) + bw[...])
    out[...] = h2 * gg * w


def _tc_node_msg(edge0, n_edges, gs, gd, eft, snwt, w1s, w1e, w1d, b1, w2, b2,
                 g1s, g1e, g1d, c1, g2, c2, ww, bw):
    BN = 1280
    grid = n_edges // BN
    blk0 = edge0 // BN
    row = lambda i: (i, 0)
    col = lambda i: (0, i + blk0)
    full = lambda i: (0, 0)

    def wspec(a):
        return pl.BlockSpec(a.shape, full)

    return pl.pallas_call(
        _node_msg_body,
        grid=(grid,),
        in_specs=[
            pl.BlockSpec((BN, DN), row),
            pl.BlockSpec((BN, DN), row),
            pl.BlockSpec((DE, BN), col),
            pl.BlockSpec((DW, BN), col),
            wspec(w1s), wspec(w1e), wspec(w1d), wspec(b1),
            wspec(w2), wspec(b2),
            wspec(g1s), wspec(g1e), wspec(g1d), wspec(c1),
            wspec(g2), wspec(c2), wspec(ww), wspec(bw),
        ],
        out_specs=pl.BlockSpec((BN, DN), row),
        out_shape=jax.ShapeDtypeStruct((n_edges, DN), jnp.float32),
        compiler_params=pltpu.CompilerParams(
            dimension_semantics=("arbitrary",)),
    )(gs, gd, eft, snwt, w1s, w1e, w1d, b1, w2, b2,
      g1s, g1e, g1d, c1, g2, c2, ww, bw)


def _node_out_body(nf, a0, a1, ew1s, eg1s, ew1d, eg1d, nn_out, te_out):
    f32 = jnp.float32
    nn = nf[...] + a0[...] + a1[...]
    nn_out[...] = nn
    te_out[...] = jnp.concatenate(
        [jnp.dot(nn, ew1s[...], preferred_element_type=f32),
         jnp.dot(nn, eg1s[...], preferred_element_type=f32),
         jnp.dot(nn, ew1d[...], preferred_element_type=f32),
         jnp.dot(nn, eg1d[...], preferred_element_type=f32)], axis=1)


def _tc_node_out(nf, a0, a1, ew1s, eg1s, ew1d, eg1d):
    BR = 1000
    grid = NN // BR
    row = lambda i: (i, 0)
    full = lambda i: (0, 0)

    def wspec(a):
        return pl.BlockSpec(a.shape, full)

    return pl.pallas_call(
        _node_out_body,
        grid=(grid,),
        in_specs=[
            pl.BlockSpec((BR, DN), row),
            pl.BlockSpec((BR, DN), row),
            pl.BlockSpec((BR, DN), row),
            wspec(ew1s), wspec(eg1s), wspec(ew1d), wspec(eg1d),
        ],
        out_specs=[
            pl.BlockSpec((BR, DN), row),
            pl.BlockSpec((BR, DN), row),
        ],
        out_shape=[
            jax.ShapeDtypeStruct((NN, DN), jnp.float32),
            jax.ShapeDtypeStruct((NN, DN), jnp.float32),
        ],
        compiler_params=pltpu.CompilerParams(
            dimension_semantics=("arbitrary",)),
    )(nf, a0, a1, ew1s, eg1s, ew1d, eg1d)


def _edge_out_body(ge, eft, sewt, ew1e, eg1e, eb1, eg1, ew2, eb2,
                   eg2, eg2b, eww, ebw, out):
    f32 = jnp.float32
    et = eft[...]
    g = ge[...]
    pre_h = (g[:, :32] + g[:, 64:96] + _dot0(et, ew1e[...]) + eb1[...])
    pre_g = (g[:, 32:64] + g[:, 96:128] + _dot0(et, eg1e[...]) + eg1[...])
    h2 = _silu(jnp.dot(_silu(pre_h), ew2[...], preferred_element_type=f32) + eb2[...])
    g2 = jax.nn.sigmoid(jnp.dot(_silu(pre_g), eg2[...], preferred_element_type=f32) + eg2b[...])
    ew = jax.nn.sigmoid(_dot0(sewt[...], eww[...]) + ebw[...])
    msg = h2 * g2 * ew
    out[...] = et + jnp.swapaxes(msg, 0, 1)


def _tc_edge_out(edge0, n_edges, ge, eft, sewt, ew1e, eg1e, eb1, eg1,
                 ew2, eb2, eg2, eg2b, eww, ebw):
    BN = 2560
    grid = n_edges // BN
    blk0 = edge0 // BN
    row = lambda i: (i, 0)
    col = lambda i: (0, i + blk0)
    ocol = lambda i: (0, i)
    full = lambda i: (0, 0)

    def wspec(a):
        return pl.BlockSpec(a.shape, full)

    return pl.pallas_call(
        _edge_out_body,
        grid=(grid,),
        in_specs=[
            pl.BlockSpec((BN, DN), row),
            pl.BlockSpec((DE, BN), col),
            pl.BlockSpec((DW, BN), col),
            wspec(ew1e), wspec(eg1e), wspec(eb1), wspec(eg1),
            wspec(ew2), wspec(eb2), wspec(eg2), wspec(eg2b),
            wspec(eww), wspec(ebw),
        ],
        out_specs=pl.BlockSpec((DE, BN), ocol),
        out_shape=jax.ShapeDtypeStruct((DE, n_edges), jnp.float32),
        compiler_params=pltpu.CompilerParams(
            dimension_semantics=("arbitrary",)),
    )(ge, eft, sewt, ew1e, eg1e, eb1, eg1, ew2, eb2, eg2, eg2b, eww, ebw)


def kernel(node_features, edge_features, edge_index, shared_node_weights,
           shared_edge_weights, nW1, nb1, nW2, nb2, nG1, ng1, nG2, ng2,
           nWw, nbw, eW1, eb1, eW2, eb2, eG1, eg1, eG2, eg2, eWw, ebw):
    f32 = jnp.float32
    src = edge_index[0]
    dst = edge_index[1]
    eft = jnp.swapaxes(edge_features, 0, 1)
    snwt = jnp.swapaxes(shared_node_weights, 0, 1)
    sewt = jnp.swapaxes(shared_edge_weights, 0, 1)

    contribs = []
    for k, (r0, nr) in enumerate(SLICES):
        gs3, gd3 = _sc_gathers[k](node_features, src, dst)
        contrib_k = _tc_node_msg(
            r0 * CH, nr * CH,
            jnp.reshape(gs3, (nr * CH, DN)), jnp.reshape(gd3, (nr * CH, DN)),
            eft, snwt,
            nW1[:DN], nW1[DN:DN + DE], nW1[DN + DE:], nb1.reshape(1, DN),
            nW2, nb2.reshape(1, DN),
            nG1[:DN], nG1[DN:DN + DE], nG1[DN + DE:], ng1.reshape(1, DN),
            nG2, ng2.reshape(1, DN), nWw, nbw.reshape(1, DN))
        contribs.append(jnp.reshape(contrib_k, (nr, CH, DN)))

    aggp = jnp.zeros((NC, NN, DN), f32)
    for g, grp in enumerate(SCATTER_GROUPS):
        aggp = _sc_scatters[g](*[contribs[k] for k in grp], dst, aggp)

    nn, te = _tc_node_out(node_features, aggp[0], aggp[1],
                          eW1[:DN], eG1[:DN], eW1[DN + DE:], eG1[DN + DE:])

    ne_parts = []
    for k, (r0, nr) in enumerate(SLICES):
        ge3 = _sc_gathers_pack[k](te, src, dst)
        ne_parts.append(_tc_edge_out(
            r0 * CH, nr * CH,
            jnp.reshape(ge3, (nr * CH, DN)),
            eft, sewt,
            eW1[DN:DN + DE], eG1[DN:DN + DE], eb1.reshape(1, 32),
            eg1.reshape(1, 32), eW2, eb2.reshape(1, DE), eG2,
            eg2.reshape(1, DE), eWw, ebw.reshape(1, DE)))

    new_edges_t = jnp.concatenate(ne_parts, axis=1)
    return nn, jnp.swapaxes(new_edges_t, 0, 1)


# speedup vs baseline: 4.1481x; 1.0097x over previous
import functools

import jax
import jax.numpy as jnp
from jax import lax
from jax.experimental import pallas as pl
from jax.experimental.pallas import tpu as pltpu
from jax.experimental.pallas import tpu_sc as plsc

NN = 10000
NE = 320000
DN = 128
DE = 16
DW = 32

NC = 2
NS = 16
NW = NC * NS
CH = 128
NROW = NE // CH
ROWS_BASE = NROW // NW
ROWS_EXTRA = NROW % NW


def _silu(x):
    return x * jax.nn.sigmoid(x)


SLICES = ((0, 320), (320, 320), (640, 640), (1280, 640), (1920, 580))
SCATTER_GROUPS = ((0, 1), (2,), (3,), (4,))


def _make_sc_gather(width, row0, nrows):
    mesh = plsc.VectorSubcoreMesh(core_axis_name="c", subcore_axis_name="s")
    rows_base = nrows // NW
    rows_extra = nrows % NW
    assert rows_base % 2 == 0

    @functools.partial(
        pl.kernel,
        mesh=mesh,
        out_type=(
            jax.ShapeDtypeStruct((nrows, CH, width), jnp.float32),
            jax.ShapeDtypeStruct((nrows, CH, width), jnp.float32),
        ),
        scratch_types=[
            pltpu.VMEM((2 * CH,), jnp.int32),
            pltpu.VMEM((2 * CH,), jnp.int32),
            pltpu.VMEM((2, CH, width), jnp.float32),
            pltpu.VMEM((2, CH, width), jnp.float32),
            pltpu.SemaphoreType.DMA,
            pltpu.SemaphoreType.DMA,
        ],
    )
    def gather_k(tab_hbm, ei_hbm, gs_hbm, gd_hbm,
                 idxs, idxd, rs, rd, gsem, wsem):
        c = lax.axis_index("c")
        s = lax.axis_index("s")
        wid = s * NC + c
        base = rows_base * wid + jnp.minimum(wid, rows_extra)

        def do_pair(row, first):
            grow = row0 + row
            pltpu.sync_copy(ei_hbm.at[0, pl.ds(grow * CH, 2 * CH)], idxs)
            pltpu.sync_copy(ei_hbm.at[1, pl.ds(grow * CH, 2 * CH)], idxd)
            @pl.when(jnp.logical_not(first))
            def _():
                pltpu.make_async_copy(rs, gs_hbm.at[pl.ds(row, 2)], wsem).wait()
                pltpu.make_async_copy(rd, gd_hbm.at[pl.ds(row, 2)], wsem).wait()
            d0 = pltpu.async_copy(tab_hbm.at[idxs.at[pl.ds(0, CH)]], rs.at[0], gsem)
            d1 = pltpu.async_copy(tab_hbm.at[idxs.at[pl.ds(CH, CH)]], rs.at[1], gsem)
            d2 = pltpu.async_copy(tab_hbm.at[idxd.at[pl.ds(0, CH)]], rd.at[0], gsem)
            d3 = pltpu.async_copy(tab_hbm.at[idxd.at[pl.ds(CH, CH)]], rd.at[1], gsem)
            d0.wait(); d1.wait(); d2.wait(); d3.wait()
            pltpu.async_copy(rs, gs_hbm.at[pl.ds(row, 2)], wsem)
            pltpu.async_copy(rd, gd_hbm.at[pl.ds(row, 2)], wsem)

        def body(i, _):
            do_pair(base + 2 * i, i == 0)
            return 0

        lax.fori_loop(0, rows_base // 2, body, 0)
        pltpu.make_async_copy(rs, gs_hbm.at[pl.ds(base, 2)], wsem).wait()
        pltpu.make_async_copy(rd, gd_hbm.at[pl.ds(base, 2)], wsem).wait()

        if rows_extra:
            @pl.when(wid < rows_extra)
            def _():
                row = base + rows_base
                grow = row0 + row
                pltpu.sync_copy(ei_hbm.at[0, pl.ds(grow * CH, CH)],
                                idxs.at[pl.ds(0, CH)])
                pltpu.sync_copy(ei_hbm.at[1, pl.ds(grow * CH, CH)],
                                idxd.at[pl.ds(0, CH)])
                d0 = pltpu.async_copy(tab_hbm.at[idxs.at[pl.ds(0, CH)]],
                                      rs.at[0], gsem)
                d1 = pltpu.async_copy(tab_hbm.at[idxd.at[pl.ds(0, CH)]],
                                      rd.at[0], gsem)
                d0.wait(); d1.wait()
                pltpu.sync_copy(rs.at[pl.ds(0, 1)], gs_hbm.at[pl.ds(row, 1)])
                pltpu.sync_copy(rd.at[pl.ds(0, 1)], gd_hbm.at[pl.ds(row, 1)])

    return gather_k


def _make_sc_gather_pack(row0, nrows):
    W = DN
    H = W // 2
    mesh = plsc.VectorSubcoreMesh(core_axis_name="c", subcore_axis_name="s")
    ch_base = nrows // NW
    ch_extra = nrows % NW
    assert ch_base % 2 == 0

    @functools.partial(
        pl.kernel,
        mesh=mesh,
        out_type=jax.ShapeDtypeStruct((nrows, CH, W), jnp.float32),
        scratch_types=[
            pltpu.VMEM((CH,), jnp.int32),
            pltpu.VMEM((CH,), jnp.int32),
            pltpu.VMEM((CH,), jnp.int32),
            pltpu.VMEM((CH,), jnp.int32),
            pltpu.VMEM((CH, W), jnp.float32),
            pltpu.VMEM((CH, W), jnp.float32),
            pltpu.VMEM((CH, W), jnp.float32),
            pltpu.VMEM((CH, W), jnp.float32),
            pltpu.VMEM((1, CH, W), jnp.float32),
            pltpu.VMEM((1, CH, W), jnp.float32),
            pltpu.SemaphoreType.DMA,
            pltpu.SemaphoreType.DMA,
        ],
    )
    def gather_k(tab_hbm, ei_hbm, g_hbm,
                 is0, is1, id0, id1, rs0, rs1, rd0, rd1, pk0, pk1,
                 gsem, wsem):
        c = lax.axis_index("c")
        s = lax.axis_index("s")
        wid = s * NC + c
        base = ch_base * wid + jnp.minimum(wid, ch_extra)
        isb = (is0, is1)
        idb = (id0, id1)
        rsb = (rs0, rs1)
        rdb = (rd0, rd1)
        pkb = (pk0, pk1)

        def fire(row, p):
            grow = row0 + row
            pltpu.sync_copy(ei_hbm.at[0, pl.ds(grow * CH, CH)], isb[p])
            pltpu.sync_copy(ei_hbm.at[1, pl.ds(grow * CH, CH)], idb[p])
            pltpu.async_copy(tab_hbm.at[isb[p]], rsb[p], gsem)
            pltpu.async_copy(tab_hbm.at[idb[p]], rdb[p], gsem)

        def merge_wb(row, p, first):
            pltpu.make_async_copy(tab_hbm.at[isb[p]], rsb[p], gsem).wait()
            pltpu.make_async_copy(tab_hbm.at[idb[p]], rdb[p], gsem).wait()
            @pl.when(jnp.logical_not(first))
            def _():
                pltpu.make_async_copy(pkb[p], g_hbm.at[pl.ds(row, 1)],
                                      wsem).wait()
            rs, rd, pk = rsb[p], rdb[p], pkb[p]

            def mrow(r, _):
                for k in range(H // 16):
                    pk[0, r, pl.ds(k * 16, 16)] = rs[r, pl.ds(k * 16, 16)]
                    pk[0, r, pl.ds(H + k * 16, 16)] = rd[r, pl.ds(H + k * 16, 16)]
                return 0

            lax.fori_loop(0, CH, mrow, 0)
            pltpu.async_copy(pk, g_hbm.at[pl.ds(row, 1)], wsem)

        fire(base, 0)

        def body(i, _):
            r0 = base + 2 * i
            fire(r0 + 1, 1)
            merge_wb(r0, 0, i == 0)
            @pl.when(2 * i + 2 < ch_base)
            def _():
                fire(r0 + 2, 0)
            merge_wb(r0 + 1, 1, i == 0)
            return 0

        lax.fori_loop(0, ch_base // 2, body, 0)
        pltpu.make_async_copy(pk0, g_hbm.at[pl.ds(base, 1)], wsem).wait()
        pltpu.make_async_copy(pk1, g_hbm.at[pl.ds(base, 1)], wsem).wait()

        if ch_extra:
            @pl.when(wid < ch_extra)
            def _():
                row = base + ch_base
                fire(row, 0)
                pltpu.make_async_copy(tab_hbm.at[is0], rs0, gsem).wait()
                pltpu.make_async_copy(tab_hbm.at[id0], rd0, gsem).wait()

                def mrow(r, _):
                    for k in range(H // 16):
                        pk0[0, r, pl.ds(k * 16, 16)] = rs0[r, pl.ds(k * 16, 16)]
                        pk0[0, r, pl.ds(H + k * 16, 16)] = rd0[r, pl.ds(H + k * 16, 16)]
                    return 0

                lax.fori_loop(0, CH, mrow, 0)
                pltpu.sync_copy(pk0, g_hbm.at[pl.ds(row, 1)])

    return gather_k


def _make_sc_scatter(slice_ids):
    mesh = plsc.VectorSubcoreMesh(core_axis_name="c", subcore_axis_name="s")
    ZR = 624
    my_slices = [SLICES[i] for i in slice_ids]

    @functools.partial(
        pl.kernel,
        mesh=mesh,
        out_type=jax.ShapeDtypeStruct((NC, NN, DN), jnp.float32),
        scratch_types=[
            pltpu.VMEM((CH,), jnp.int32),
            pltpu.VMEM((CH,), jnp.int32),
            pltpu.VMEM((2, CH, DN), jnp.float32),
            pltpu.VMEM_SHARED((NN, DN), jnp.float32),
            pltpu.SemaphoreType.DMA,
        ],
    )
    def scatter_k(*args):
        u_hbms = args[:len(my_slices)]
        ei_hbm, init_hbm, out_hbm, idx0, idx1, upd, acc, sem = \
            args[len(my_slices):]
        c = lax.axis_index("c")
        s = lax.axis_index("s")
        wid = s * NC + c

        pltpu.sync_copy(init_hbm.at[c, pl.ds(s * ZR, ZR)],
                        acc.at[pl.ds(s * ZR, ZR)])
        @pl.when(s == NS - 1)
        def _():
            tail = NN - NS * ZR
            pltpu.sync_copy(init_hbm.at[c, pl.ds(NS * ZR, tail)],
                            acc.at[pl.ds(NS * ZR, tail)])
        plsc.subcore_barrier()

        for u_hbm, (row0, nrows) in zip(u_hbms, my_slices):
            rows_base = nrows // NW
            rows_extra = nrows % NW
            base = rows_base * wid + jnp.minimum(wid, rows_extra)

            def body(i, _, u_hbm=u_hbm, base=base, row0=row0):
                row = base + 2 * i
                grow = row0 + row
                pltpu.sync_copy(ei_hbm.at[1, pl.ds(grow * CH, CH)], idx0)
                pltpu.sync_copy(ei_hbm.at[1, pl.ds((grow + 1) * CH, CH)], idx1)
                pltpu.sync_copy(u_hbm.at[pl.ds(row, 2)], upd)
                a0 = pltpu.async_copy(upd.at[0], acc.at[idx0], sem, add=True)
                a1 = pltpu.async_copy(upd.at[1], acc.at[idx1], sem, add=True)
                a0.wait(); a1.wait()
                return 0

            lax.fori_loop(0, rows_base // 2, body, 0)

            if rows_extra:
                @pl.when(wid < rows_extra)
                def _(u_hbm=u_hbm, base=base, row0=row0,
                      rows_base=rows_base):
                    row = base + rows_base
                    grow = row0 + row
                    pltpu.sync_copy(ei_hbm.at[1, pl.ds(grow * CH, CH)], idx0)
                    pltpu.sync_copy(u_hbm.at[pl.ds(row, 1)],
                                    upd.at[pl.ds(0, 1)])
                    pltpu.sync_copy(upd.at[0], acc.at[idx0], add=True)

        plsc.subcore_barrier()
        pltpu.sync_copy(acc.at[pl.ds(s * ZR, ZR)], out_hbm.at[c, pl.ds(s * ZR, ZR)])
        @pl.when(s == NS - 1)
        def _():
            tail = NN - NS * ZR
            pltpu.sync_copy(acc.at[pl.ds(NS * ZR, tail)],
                            out_hbm.at[c, pl.ds(NS * ZR, tail)])

    return scatter_k


_sc_gathers = tuple(_make_sc_gather(DN, r0, nr) for r0, nr in SLICES)
_sc_gathers_pack = tuple(_make_sc_gather_pack(r0, nr) for r0, nr in SLICES)
_sc_scatters = tuple(_make_sc_scatter(g) for g in SCATTER_GROUPS)


def _dot0(a, b):
    return lax.dot_general(a, b, (((0,), (0,)), ((), ())),
                           preferred_element_type=jnp.float32)


def _node_msg_body(gs, gd, eft, snwt, w1s, w1e, w1d, b1, w2, b2,
                   g1s, g1e, g1d, c1, g2, c2, ww, bw, out):
    f32 = jnp.float32
    xs = gs[...].astype(f32)
    xd = gd[...].astype(f32)
    et = eft[...]
    pre_h = (jnp.dot(xs, w1s[...], preferred_element_type=f32)
             + _dot0(et, w1e[...])
             + jnp.dot(xd, w1d[...], preferred_element_type=f32) + b1[...])
    pre_g = (jnp.dot(xs, g1s[...], preferred_element_type=f32)
             + _dot0(et, g1e[...])
             + jnp.dot(xd, g1d[...], preferred_element_type=f32) + c1[...])
    h2 = _silu(jnp.dot(_silu(pre_h), w2[...], preferred_element_type=f32) + b2[...])
    gg = jax.nn.sigmoid(jnp.dot(_silu(pre_g), g2[...], preferred_element_type=f32) + c2[...])
    w = jax.nn.sigmoid(_dot0(snwt[...], ww[...]) + bw[...])
    out[...] = h2 * gg * w


def _tc_node_msg(edge0, n_edges, gs, gd, eft, snwt, w1s, w1e, w1d, b1, w2, b2,
                 g1s, g1e, g1d, c1, g2, c2, ww, bw):
    BN = 1280
    grid = n_edges // BN
    blk0 = edge0 // BN
    row = lambda i: (i, 0)
    col = lambda i: (0, i + blk0)
    full = lambda i: (0, 0)

    def wspec(a):
        return pl.BlockSpec(a.shape, full)

    return pl.pallas_call(
        _node_msg_body,
        grid=(grid,),
        in_specs=[
            pl.BlockSpec((BN, DN), row),
            pl.BlockSpec((BN, DN), row),
            pl.BlockSpec((DE, BN), col),
            pl.BlockSpec((DW, BN), col),
            wspec(w1s), wspec(w1e), wspec(w1d), wspec(b1),
            wspec(w2), wspec(b2),
            wspec(g1s), wspec(g1e), wspec(g1d), wspec(c1),
            wspec(g2), wspec(c2), wspec(ww), wspec(bw),
        ],
        out_specs=pl.BlockSpec((BN, DN), row),
        out_shape=jax.ShapeDtypeStruct((n_edges, DN), jnp.float32),
        compiler_params=pltpu.CompilerParams(
            dimension_semantics=("arbitrary",)),
    )(gs, gd, eft, snwt, w1s, w1e, w1d, b1, w2, b2,
      g1s, g1e, g1d, c1, g2, c2, ww, bw)


def _node_out_body(nf, a0, a1, ew1s, eg1s, ew1d, eg1d, nn_out, te_out):
    f32 = jnp.float32
    nn = nf[...] + a0[...] + a1[...]
    nn_out[...] = nn
    te_out[...] = jnp.concatenate(
        [jnp.dot(nn, ew1s[...], preferred_element_type=f32),
         jnp.dot(nn, eg1s[...], preferred_element_type=f32),
         jnp.dot(nn, ew1d[...], preferred_element_type=f32),
         jnp.dot(nn, eg1d[...], preferred_element_type=f32)], axis=1)


def _tc_node_out(nf, a0, a1, ew1s, eg1s, ew1d, eg1d):
    BR = 1000
    grid = NN // BR
    row = lambda i: (i, 0)
    full = lambda i: (0, 0)

    def wspec(a):
        return pl.BlockSpec(a.shape, full)

    return pl.pallas_call(
        _node_out_body,
        grid=(grid,),
        in_specs=[
            pl.BlockSpec((BR, DN), row),
            pl.BlockSpec((BR, DN), row),
            pl.BlockSpec((BR, DN), row),
            wspec(ew1s), wspec(eg1s), wspec(ew1d), wspec(eg1d),
        ],
        out_specs=[
            pl.BlockSpec((BR, DN), row),
            pl.BlockSpec((BR, DN), row),
        ],
        out_shape=[
            jax.ShapeDtypeStruct((NN, DN), jnp.float32),
            jax.ShapeDtypeStruct((NN, DN), jnp.float32),
        ],
        compiler_params=pltpu.CompilerParams(
            dimension_semantics=("arbitrary",)),
    )(nf, a0, a1, ew1s, eg1s, ew1d, eg1d)


def _edge_out_body(ge, eft, sewt, ew1e, eg1e, eb1, eg1, ew2, eb2,
                   eg2, eg2b, eww, ebw, out):
    f32 = jnp.float32
    et = eft[...]
    g = ge[...]
    pre_h = (g[:, :32] + g[:, 64:96] + _dot0(et, ew1e[...]) + eb1[...])
    pre_g = (g[:, 32:64] + g[:, 96:128] + _dot0(et, eg1e[...]) + eg1[...])
    h2 = _silu(jnp.dot(_silu(pre_h), ew2[...], preferred_element_type=f32) + eb2[...])
    g2 = jax.nn.sigmoid(jnp.dot(_silu(pre_g), eg2[...], preferred_element_type=f32) + eg2b[...])
    ew = jax.nn.sigmoid(_dot0(sewt[...], eww[...]) + ebw[...])
    msg = h2 * g2 * ew
    out[...] = et + jnp.swapaxes(msg, 0, 1)


def _tc_edge_out(edge0, n_edges, ge, eft, sewt, ew1e, eg1e, eb1, eg1,
                 ew2, eb2, eg2, eg2b, eww, ebw):
    BN = 2560
    grid = n_edges // BN
    blk0 = edge0 // BN
    row = lambda i: (i, 0)
    col = lambda i: (0, i + blk0)
    ocol = lambda i: (0, i)
    full = lambda i: (0, 0)

    def wspec(a):
        return pl.BlockSpec(a.shape, full)

    return pl.pallas_call(
        _edge_out_body,
        grid=(grid,),
        in_specs=[
            pl.BlockSpec((BN, DN), row),
            pl.BlockSpec((DE, BN), col),
            pl.BlockSpec((DW, BN), col),
            wspec(ew1e), wspec(eg1e), wspec(eb1), wspec(eg1),
            wspec(ew2), wspec(eb2), wspec(eg2), wspec(eg2b),
            wspec(eww), wspec(ebw),
        ],
        out_specs=pl.BlockSpec((DE, BN), ocol),
        out_shape=jax.ShapeDtypeStruct((DE, n_edges), jnp.float32),
        compiler_params=pltpu.CompilerParams(
            dimension_semantics=("arbitrary",)),
    )(ge, eft, sewt, ew1e, eg1e, eb1, eg1, ew2, eb2, eg2, eg2b, eww, ebw)


def kernel(node_features, edge_features, edge_index, shared_node_weights,
           shared_edge_weights, nW1, nb1, nW2, nb2, nG1, ng1, nG2, ng2,
           nWw, nbw, eW1, eb1, eW2, eb2, eG1, eg1, eG2, eg2, eWw, ebw):
    f32 = jnp.float32
    eft = jnp.swapaxes(edge_features, 0, 1)
    snwt = jnp.swapaxes(shared_node_weights, 0, 1)
    sewt = jnp.swapaxes(shared_edge_weights, 0, 1)

    contribs = []
    for k, (r0, nr) in enumerate(SLICES):
        gs3, gd3 = _sc_gathers[k](node_features, edge_index)
        contrib_k = _tc_node_msg(
            r0 * CH, nr * CH,
            jnp.reshape(gs3, (nr * CH, DN)), jnp.reshape(gd3, (nr * CH, DN)),
            eft, snwt,
            nW1[:DN], nW1[DN:DN + DE], nW1[DN + DE:], nb1.reshape(1, DN),
            nW2, nb2.reshape(1, DN),
            nG1[:DN], nG1[DN:DN + DE], nG1[DN + DE:], ng1.reshape(1, DN),
            nG2, ng2.reshape(1, DN), nWw, nbw.reshape(1, DN))
        contribs.append(jnp.reshape(contrib_k, (nr, CH, DN)))

    aggp = jnp.zeros((NC, NN, DN), f32)
    for g, grp in enumerate(SCATTER_GROUPS):
        aggp = _sc_scatters[g](*[contribs[k] for k in grp], edge_index, aggp)

    nn, te = _tc_node_out(node_features, aggp[0], aggp[1],
                          eW1[:DN], eG1[:DN], eW1[DN + DE:], eG1[DN + DE:])

    ne_parts = []
    for k, (r0, nr) in enumerate(SLICES):
        ge3 = _sc_gathers_pack[k](te, edge_index)
        ne_parts.append(_tc_edge_out(
            r0 * CH, nr * CH,
            jnp.reshape(ge3, (nr * CH, DN)),
            eft, sewt,
            eW1[DN:DN + DE], eG1[DN:DN + DE], eb1.reshape(1, 32),
            eg1.reshape(1, 32), eW2, eb2.reshape(1, DE), eG2,
            eg2.reshape(1, DE), eWw, ebw.reshape(1, DE)))

    new_edges_t = jnp.concatenate(ne_parts, axis=1)
    return nn, jnp.swapaxes(new_edges_t, 0, 1)
